# baseline (speedup 1.0000x reference)
import functools

import jax
import jax.numpy as jnp
from jax import lax
from jax.experimental import pallas as pl
from jax.experimental.pallas import tpu as pltpu
from jax.experimental.pallas import tpu_sc as plsc

N = 126976
E = 262144
DIN = 128
HID = 30
DP = 32
FH = 16
NGRAPH = 2048
NCLS = 3
WP = 128

NC, NS, L = 2, 16, 16
HALF = N // 2
DUMP = 1024
ACC_ROWS = HALF + DUMP
EPT_M = E // NS
CH = 512
NCHUNK = EPT_M // CH
NOCT = N // 8

_MESH = plsc.VectorSubcoreMesh(core_axis_name="c", subcore_axis_name="s")
_SC_PARAMS = pltpu.CompilerParams(use_tc_tiling_on_sc=False)


def _zero_rows16(ref, nrows):
    zv = jnp.zeros((L,), jnp.float32)

    def body(i, _):
        ref[i, pl.ds(0, L)] = zv
        return 0

    lax.fori_loop(0, nrows, body, 0)


def _zero_acc(acc, s, zeros_hbm):
    rows_per_tile = ACC_ROWS // NS
    pltpu.sync_copy(zeros_hbm, acc.at[pl.ds(s * rows_per_tile, rows_per_tile)])


def _write_half(acc, eout, c, s, bounce):
    del bounce
    rpt = HALF // NS
    o = s * rpt
    pltpu.sync_copy(acc.at[pl.ds(o, rpt)], eout.at[pl.ds(c * HALF + o, rpt)])


def _deg_body(col_hbm, wrep_hbm, zer_hbm, degrep, wrep, colbuf, dstbuf, zbuf,
              acc, sem):
    c = lax.axis_index("c")
    s = lax.axis_index("s")
    lo = c * HALF

    pltpu.sync_copy(wrep_hbm, wrep)
    _zero_acc(acc, s, zer_hbm)
    plsc.subcore_barrier()

    def chunk(ch, _):
        ebase = s * EPT_M + ch * 2048
        pltpu.sync_copy(col_hbm.at[pl.ds(ebase, 2048)], colbuf)

        def grp(g, _):
            col_v = colbuf[pl.ds(g * L, L)]
            inhalf = (col_v >= lo) & (col_v < lo + HALF)
            dst = jnp.where(inhalf, col_v - lo, HALF + (col_v & (DUMP - 1)))
            dstbuf[pl.ds(g * L, L)] = dst
            return 0

        lax.fori_loop(0, 2048 // L, grp, 0)
        pltpu.sync_copy(wrep, acc.at[dstbuf], add=True)
        return 0

    lax.fori_loop(0, EPT_M // 2048, chunk, 0)
    plsc.subcore_barrier()
    _write_half(acc, degrep, c, s, zbuf)


_deg_call = functools.partial(
    pl.kernel,
    out_type=jax.ShapeDtypeStruct((N, FH), jnp.float32),
    mesh=_MESH,
    compiler_params=_SC_PARAMS,
    scratch_types=[
        pltpu.VMEM((2048, FH), jnp.float32),
        pltpu.VMEM((2048,), jnp.int32),
        pltpu.VMEM((2048,), jnp.int32),
        pltpu.VMEM((8, FH), jnp.float32),
        pltpu.VMEM_SHARED((ACC_ROWS, FH), jnp.float32),
        pltpu.SemaphoreType.DMA,
    ],
)(_deg_body)


def _hop_body(row_hbm, col_hbm, w16_hbm, zer_hbm, ga_hbm, gb_hbm,
              eout_a, eout_b,
              w16, rowall, colbuf, dstall, gb0, gb1, gb2, acc,
              gs0, gs1, gs2, ss0, ss1, ss2):
    c = lax.axis_index("c")
    s = lax.axis_index("s")
    lo = c * HALF
    gbufs = (gb0, gb1, gb2)
    gsems = (gs0, gs1, gs2)
    ssems = (ss0, ss1, ss2)

    pltpu.sync_copy(w16_hbm, w16)
    ebase = s * EPT_M
    pltpu.sync_copy(row_hbm.at[pl.ds(ebase, EPT_M)], rowall)

    def dchunk(dc, _):
        pltpu.sync_copy(col_hbm.at[pl.ds(ebase + dc * 2048, 2048)], colbuf)

        def dstg(g, _):
            col_v = colbuf[pl.ds(g * L, L)]
            inhalf = (col_v >= lo) & (col_v < lo + HALF)
            dst = jnp.where(inhalf, col_v - lo, HALF + (col_v & (DUMP - 1)))
            dstall[dc * (2048 // CH) + g // (CH // L),
                   pl.ds((g % (CH // L)) * L, L)] = dst
            return 0

        lax.fori_loop(0, 2048 // L, dstg, 0)
        return 0

    lax.fori_loop(0, EPT_M // 2048, dchunk, 0)

    for fh in range(2):
        g_hbm = ga_hbm if fh == 0 else gb_hbm
        eout = eout_a if fh == 0 else eout_b

        _zero_acc(acc, s, zer_hbm)
        plsc.subcore_barrier()

        def start_gather(chv, b):
            pltpu.async_copy(g_hbm.at[rowall.at[pl.ds(chv * CH, CH)]],
                             gbufs[b], gsems[b])

        def wait_slot(sems, b):
            pltpu.make_async_copy(g_hbm.at[pl.ds(0, CH)], gbufs[b],
                                  sems[b]).wait()

        def start_scatter(chv, b):
            pltpu.async_copy(gbufs[b], acc.at[dstall.at[chv]],
                             ssems[b], add=True)

        def scale(gbuf):
            def grp(g, _):
                off = g * L
                wb = (g & 7) * (L * L)
                for k in range(L):
                    gbuf[off + k, pl.ds(0, L)] = (
                        gbuf[off + k, pl.ds(0, L)] * w16[pl.ds(wb + k * L, L)])
                return 0

            lax.fori_loop(0, CH // L, grp, 0)

        start_gather(0, 0)
        start_gather(1, 1)

        def tri(t, _):
            for j in range(3):
                chv = t * 3 + j
                b2 = (j + 2) % 3
                wait_slot(gsems, j)
                scale(gbufs[j])
                start_scatter(chv, j)
                if j == 0:
                    @pl.when(t >= 1)
                    def _():
                        wait_slot(ssems, b2)
                else:
                    wait_slot(ssems, b2)
                start_gather(chv + 2, b2)
            return 0

        lax.fori_loop(0, NCHUNK // 3, tri, 0)
        for chv in (NCHUNK - 2, NCHUNK - 1):
            j = chv % 3
            wait_slot(gsems, j)
            scale(gbufs[j])
            start_scatter(chv, j)
            wait_slot(ssems, (j + 2) % 3)
        wait_slot(ssems, (NCHUNK - 1) % 3)
        plsc.subcore_barrier()

        _write_half(acc, eout, c, s, gb0)
        plsc.subcore_barrier()


_hop_call = functools.partial(
    pl.kernel,
    out_type=[jax.ShapeDtypeStruct((N, FH), jnp.float32),
              jax.ShapeDtypeStruct((N, FH), jnp.float32)],
    mesh=_MESH,
    compiler_params=_SC_PARAMS,
    scratch_types=[
        pltpu.VMEM((WP * L,), jnp.float32),
        pltpu.VMEM((EPT_M,), jnp.int32),
        pltpu.VMEM((2048,), jnp.int32),
        pltpu.VMEM((NCHUNK, CH), jnp.int32),
        pltpu.VMEM((CH, FH), jnp.float32),
        pltpu.VMEM((CH, FH), jnp.float32),
        pltpu.VMEM((CH, FH), jnp.float32),
        pltpu.VMEM_SHARED((ACC_ROWS, FH), jnp.float32),
        pltpu.SemaphoreType.DMA,
        pltpu.SemaphoreType.DMA,
        pltpu.SemaphoreType.DMA,
        pltpu.SemaphoreType.DMA,
        pltpu.SemaphoreType.DMA,
        pltpu.SemaphoreType.DMA,
    ],
)(_hop_body)


def _pool_body(batch_hbm, za_hbm, zb_hbm, pout, zrow, bbuf, acc, sem):
    c = lax.axis_index("c")
    s = lax.axis_index("s")
    rpt = NGRAPH // NS
    rows = N // (NC * NS)

    for fh in range(2):
        z_hbm = za_hbm if fh == 0 else zb_hbm

        _zero_rows16(zrow, rpt)
        pltpu.sync_copy(zrow.at[pl.ds(0, rpt)], acc.at[pl.ds(s * rpt, rpt)])
        plsc.subcore_barrier()

        def chunk(ch, _):
            base = (c * NS + s) * rows + ch * 496
            pltpu.sync_copy(batch_hbm.at[pl.ds(base, 496)], bbuf)
            pltpu.sync_copy(z_hbm.at[pl.ds(base, 496)], zrow)
            pltpu.sync_copy(zrow, acc.at[bbuf], add=True)
            return 0

        lax.fori_loop(0, rows // 496, chunk, 0)
        plsc.subcore_barrier()

        pltpu.sync_copy(acc.at[pl.ds(s * rpt, rpt)], zrow.at[pl.ds(0, rpt)])
        pltpu.sync_copy(zrow.at[pl.ds(0, rpt)],
                        pout.at[pl.ds((c * 2 + fh) * NGRAPH + s * rpt, rpt)])
        plsc.subcore_barrier()


_pool_call = functools.partial(
    pl.kernel,
    out_type=jax.ShapeDtypeStruct((NC * 2 * NGRAPH, FH), jnp.float32),
    mesh=_MESH,
    compiler_params=_SC_PARAMS,
    scratch_types=[
        pltpu.VMEM((496, FH), jnp.float32),
        pltpu.VMEM((496,), jnp.int32),
        pltpu.VMEM_SHARED((NGRAPH, FH), jnp.float32),
        pltpu.SemaphoreType.DMA,
    ],
)(_pool_body)


OBLK = 512
NOBLK = NOCT // OBLK


def _dis_from(degrep):
    deg = degrep + 1.0
    return jnp.where(deg > 0, lax.rsqrt(deg), 0.0)


def _pro_body(x3_ref, wt_ref, dg_ref, ua_ref, ub_ref):
    wt = wt_ref[...]
    ys = [jnp.dot(x3_ref[:, j, :], wt, preferred_element_type=jnp.float32)
          for j in range(8)]
    ya = jnp.concatenate([y[:, :FH] for y in ys], axis=1)
    yb = jnp.concatenate([y[:, FH:] for y in ys], axis=1)
    dis = _dis_from(dg_ref[...])
    ua_ref[...] = ya * dis
    ub_ref[...] = yb * dis


def _prologue(x3, wt, degrep):
    o16 = pl.BlockSpec((OBLK, 128), lambda i: (i, 0))
    return pl.pallas_call(
        _pro_body,
        grid=(NOBLK,),
        in_specs=[
            pl.BlockSpec((OBLK, 8, 128), lambda i: (i, 0, 0)),
            pl.BlockSpec((DIN, DP), lambda i: (0, 0)),
            o16,
        ],
        out_specs=[o16, o16],
        out_shape=[
            jax.ShapeDtypeStruct((NOCT, 128), jnp.float32),
            jax.ShapeDtypeStruct((NOCT, 128), jnp.float32),
        ],
    )(x3, wt, degrep)


def _mid_body(ea_ref, eb_ref, ua_ref, ub_ref, dg_ref, wa_ref, wb_ref):
    deg = dg_ref[...] + 1.0
    d2 = jnp.where(deg > 0, 1.0 / deg, 0.0)
    wa_ref[...] = d2 * (ea_ref[...] + ua_ref[...])
    wb_ref[...] = d2 * (eb_ref[...] + ub_ref[...])


def _mid(ea, eb, ua, ub, degrep):
    o16 = pl.BlockSpec((OBLK, 128), lambda i: (i, 0))
    return pl.pallas_call(
        _mid_body,
        grid=(NOBLK,),
        in_specs=[o16, o16, o16, o16, o16],
        out_specs=[o16, o16],
        out_shape=[jax.ShapeDtypeStruct((NOCT, 128), jnp.float32),
                   jax.ShapeDtypeStruct((NOCT, 128), jnp.float32)],
    )(ea, eb, ua, ub, degrep)


def _post_body(ea_ref, eb_ref, wa_ref, wb_ref, dg_ref, ba_ref, bb_ref,
               gs_ref, r8_ref, za_ref, zb_ref):
    dis = _dis_from(dg_ref[...])
    ha = dis * (ea_ref[...] + wa_ref[...]) + ba_ref[...]
    hb = dis * (eb_ref[...] + wb_ref[...]) + bb_ref[...]
    exa = jnp.exp(jnp.minimum(ha, 85.0))
    exb = jnp.exp(jnp.minimum(hb, 85.0))
    s8 = (jnp.dot(exa, gs_ref[...], preferred_element_type=jnp.float32)
          + jnp.dot(exb, gs_ref[...], preferred_element_type=jnp.float32))
    rep = jnp.dot(1.0 / s8, r8_ref[...], preferred_element_type=jnp.float32)
    za_ref[...] = exa * rep
    zb_ref[...] = exb * rep


def _post(ea, eb, wa, wb, degrep, ba, bb, gsum, r8):
    o16 = pl.BlockSpec((OBLK, 128), lambda i: (i, 0))
    return pl.pallas_call(
        _post_body,
        grid=(NOBLK,),
        in_specs=[o16, o16, o16, o16, o16,
                  pl.BlockSpec((1, 128), lambda i: (0, 0)),
                  pl.BlockSpec((1, 128), lambda i: (0, 0)),
                  pl.BlockSpec((128, 8), lambda i: (0, 0)),
                  pl.BlockSpec((8, 128), lambda i: (0, 0))],
        out_specs=[o16, o16],
        out_shape=[jax.ShapeDtypeStruct((NOCT, 128), jnp.float32),
                   jax.ShapeDtypeStruct((NOCT, 128), jnp.float32)],
    )(ea, eb, wa, wb, degrep, ba, bb, gsum, r8)


def _fin_body(p0a_ref, p1a_ref, p0b_ref, p1b_ref, w_ref, b_ref, o_ref):
    p = jnp.concatenate([p0a_ref[...] + p1a_ref[...],
                         p0b_ref[...] + p1b_ref[...]], axis=1)
    lg = jnp.dot(p, w_ref[...], preferred_element_type=jnp.float32) + b_ref[...]
    m = jnp.max(lg, axis=-1, keepdims=True)
    ex = jnp.exp(lg - m)
    o_ref[...] = ex / jnp.sum(ex, axis=-1, keepdims=True)


def _final(p0a, p1a, p0b, p1b, fcwt, fcb):
    return pl.pallas_call(
        _fin_body,
        out_shape=jax.ShapeDtypeStruct((NGRAPH, 8), jnp.float32),
    )(p0a, p1a, p0b, p1b, fcwt, fcb)


def kernel(len_y, x, index, batch, weight, lin_W, lin_b, fc_W, fc_b):
    del len_y
    row = index[0]
    col = index[1]
    f32 = jnp.float32

    wrep = jnp.broadcast_to(jnp.tile(weight, L)[:, None], (2048, FH))
    w16 = jnp.repeat(weight, L)
    zer = jnp.zeros((ACC_ROWS // NS, FH), f32)

    degrep16 = _deg_call(col, wrep, zer)
    degrep = degrep16.reshape(NOCT, 128)

    wt = jnp.zeros((DP, DIN), f32).at[:HID].set(lin_W).T
    x3 = x.reshape(NOCT, 8, DIN)
    ua_o, ub_o = _prologue(x3, wt, degrep)

    def flat(a):
        return a.reshape(N, FH)

    def oct_(a):
        return a.reshape(NOCT, 128)

    e1a, e1b = _hop_call(row, col, w16, zer, flat(ua_o), flat(ub_o))
    wa_o, wb_o = _mid(oct_(e1a), oct_(e1b), ua_o, ub_o, degrep)
    e2a, e2b = _hop_call(row, col, w16, zer, flat(wa_o), flat(wb_o))

    lane = jnp.arange(128)
    ba = jnp.tile(lin_b[:FH], 8)
    bbpat = jnp.concatenate([lin_b[FH:HID], jnp.full((2,), -1e30, f32)])
    bb = jnp.tile(bbpat, 8)
    gsum = (lane[:, None] // FH == jnp.arange(8)[None, :]).astype(f32)
    r8 = gsum.T

    za_o, zb_o = _post(oct_(e2a), oct_(e2b), wa_o, wb_o, degrep,
                       ba.reshape(1, 128), bb.reshape(1, 128), gsum, r8)

    pools = _pool_call(batch, flat(za_o), flat(zb_o))
    p0a = pools[:NGRAPH]
    p0b = pools[NGRAPH:2 * NGRAPH]
    p1a = pools[2 * NGRAPH:3 * NGRAPH]
    p1b = pools[3 * NGRAPH:]

    fcw = jnp.zeros((8, DP), f32).at[:NCLS, :HID].set(fc_W)
    fcb = jnp.concatenate([fc_b, jnp.full((5,), -1e30, f32)])
    out8 = _final(p0a, p1a, p0b, p1b, fcw.T, fcb.reshape(1, 8))
    return out8[:, :NCLS]


# speedup vs baseline: 1.2752x; 1.2752x over previous
import functools

import jax
import jax.numpy as jnp
from jax import lax
from jax.experimental import pallas as pl
from jax.experimental.pallas import tpu as pltpu
from jax.experimental.pallas import tpu_sc as plsc

N = 126976
E = 262144
DIN = 128
HID = 30
DP = 32
FH = 16
NGRAPH = 2048
NCLS = 3
WP = 128

NC, NS, L = 2, 16, 16
HALF = N // 2
DUMP = 1024
ACC_ROWS = HALF + DUMP
EPT_M = E // NS
CH = 256
NCHUNK = EPT_M // CH
NOCT = N // 8

_MESH = plsc.VectorSubcoreMesh(core_axis_name="c", subcore_axis_name="s")
_SC_PARAMS = pltpu.CompilerParams(use_tc_tiling_on_sc=False)


def _zero_rows16(ref, nrows):
    zv = jnp.zeros((L,), jnp.float32)

    def body(i, _):
        ref[i, pl.ds(0, L)] = zv
        return 0

    lax.fori_loop(0, nrows, body, 0)


def _zero_acc(acc, s, zeros_hbm):
    rows_per_tile = ACC_ROWS // NS
    pltpu.sync_copy(zeros_hbm, acc.at[pl.ds(s * rows_per_tile, rows_per_tile)])


def _write_half(acc, eout, c, s, bounce):
    del bounce
    rpt = HALF // NS
    o = s * rpt
    pltpu.sync_copy(acc.at[pl.ds(o, rpt)], eout.at[pl.ds(c * HALF + o, rpt)])


def _deg_body(col_hbm, wrep_hbm, zer_hbm, degrep, wrep, colbuf, dstbuf, zbuf,
              acc, sem):
    c = lax.axis_index("c")
    s = lax.axis_index("s")
    lo = c * HALF

    pltpu.sync_copy(wrep_hbm, wrep)
    _zero_acc(acc, s, zer_hbm)
    plsc.subcore_barrier()

    def chunk(ch, _):
        ebase = s * EPT_M + ch * 2048
        pltpu.sync_copy(col_hbm.at[pl.ds(ebase, 2048)], colbuf)

        def grp(g, _):
            col_v = colbuf[pl.ds(g * L, L)]
            inhalf = (col_v >= lo) & (col_v < lo + HALF)
            dst = jnp.where(inhalf, col_v - lo, HALF + (col_v & (DUMP - 1)))
            dstbuf[pl.ds(g * L, L)] = dst
            return 0

        lax.fori_loop(0, 2048 // L, grp, 0)
        pltpu.sync_copy(wrep, acc.at[dstbuf], add=True)
        return 0

    lax.fori_loop(0, EPT_M // 2048, chunk, 0)
    plsc.subcore_barrier()
    _write_half(acc, degrep, c, s, zbuf)


_deg_call = functools.partial(
    pl.kernel,
    out_type=jax.ShapeDtypeStruct((N, FH), jnp.float32),
    mesh=_MESH,
    compiler_params=_SC_PARAMS,
    scratch_types=[
        pltpu.VMEM((2048, FH), jnp.float32),
        pltpu.VMEM((2048,), jnp.int32),
        pltpu.VMEM((2048,), jnp.int32),
        pltpu.VMEM((8, FH), jnp.float32),
        pltpu.VMEM_SHARED((ACC_ROWS, FH), jnp.float32),
        pltpu.SemaphoreType.DMA,
    ],
)(_deg_body)


def _hop_body(row_hbm, col_hbm, w16_hbm, zer_hbm, ga_hbm, gb_hbm,
              eout_a, eout_b,
              w16, rowall, colbuf, dstall, gb0, gb1, gb2, gb3, acc,
              gs0, gs1, gs2, gs3, ss0, ss1, ss2, ss3):
    c = lax.axis_index("c")
    s = lax.axis_index("s")
    lo = c * HALF
    gbufs = (gb0, gb1, gb2, gb3)
    gsems = (gs0, gs1, gs2, gs3)
    ssems = (ss0, ss1, ss2, ss3)

    pltpu.sync_copy(w16_hbm, w16)
    ebase = s * EPT_M
    pltpu.sync_copy(row_hbm.at[pl.ds(ebase, EPT_M)], rowall)

    def dchunk(dc, _):
        pltpu.sync_copy(col_hbm.at[pl.ds(ebase + dc * 2048, 2048)], colbuf)

        def dstg(g, _):
            col_v = colbuf[pl.ds(g * L, L)]
            inhalf = (col_v >= lo) & (col_v < lo + HALF)
            dst = jnp.where(inhalf, col_v - lo, HALF + (col_v & (DUMP - 1)))
            dstall[dc * (2048 // CH) + g // (CH // L),
                   pl.ds((g % (CH // L)) * L, L)] = dst
            return 0

        lax.fori_loop(0, 2048 // L, dstg, 0)
        return 0

    lax.fori_loop(0, EPT_M // 2048, dchunk, 0)

    for fh in range(2):
        g_hbm = ga_hbm if fh == 0 else gb_hbm
        eout = eout_a if fh == 0 else eout_b

        _zero_acc(acc, s, zer_hbm)
        plsc.subcore_barrier()

        def start_gather(chv, b):
            pltpu.async_copy(g_hbm.at[rowall.at[pl.ds(chv * CH, CH)]],
                             gbufs[b], gsems[b])

        def wait_slot(sems, b):
            pltpu.make_async_copy(g_hbm.at[pl.ds(0, CH)], gbufs[b],
                                  sems[b]).wait()

        def start_scatter(chv, b):
            pltpu.async_copy(gbufs[b], acc.at[dstall.at[chv]],
                             ssems[b], add=True)

        def scale(gbuf):
            @plsc.parallel_loop(0, CH // L, unroll=2)
            def _(g):
                off = g * L
                wb = (g & 7) * (L * L)
                for k in range(L):
                    gbuf[off + k, pl.ds(0, L)] = (
                        gbuf[off + k, pl.ds(0, L)] * w16[pl.ds(wb + k * L, L)])

        start_gather(0, 0)
        start_gather(1, 1)

        def quad(q, _):
            for j in range(4):
                chv = q * 4 + j
                b2 = (j + 2) % 4
                wait_slot(gsems, j)
                scale(gbufs[j])
                start_scatter(chv, j)
                if j < 2:
                    @pl.when(q >= 1)
                    def _():
                        wait_slot(ssems, b2)
                    start_gather(chv + 2, b2)
                else:
                    wait_slot(ssems, b2)

                    @pl.when(q < NCHUNK // 4 - 1)
                    def _():
                        start_gather(chv + 2, b2)
            return 0

        lax.fori_loop(0, NCHUNK // 4, quad, 0)
        wait_slot(ssems, 2)
        wait_slot(ssems, 3)
        plsc.subcore_barrier()

        _write_half(acc, eout, c, s, gb0)
        plsc.subcore_barrier()


_hop_call = functools.partial(
    pl.kernel,
    out_type=[jax.ShapeDtypeStruct((N, FH), jnp.float32),
              jax.ShapeDtypeStruct((N, FH), jnp.float32)],
    mesh=_MESH,
    compiler_params=_SC_PARAMS,
    scratch_types=[
        pltpu.VMEM((WP * L,), jnp.float32),
        pltpu.VMEM((EPT_M,), jnp.int32),
        pltpu.VMEM((2048,), jnp.int32),
        pltpu.VMEM((NCHUNK, CH), jnp.int32),
        pltpu.VMEM((CH, FH), jnp.float32),
        pltpu.VMEM((CH, FH), jnp.float32),
        pltpu.VMEM((CH, FH), jnp.float32),
        pltpu.VMEM((CH, FH), jnp.float32),
        pltpu.VMEM_SHARED((ACC_ROWS, FH), jnp.float32),
        pltpu.SemaphoreType.DMA,
        pltpu.SemaphoreType.DMA,
        pltpu.SemaphoreType.DMA,
        pltpu.SemaphoreType.DMA,
        pltpu.SemaphoreType.DMA,
        pltpu.SemaphoreType.DMA,
        pltpu.SemaphoreType.DMA,
        pltpu.SemaphoreType.DMA,
    ],
)(_hop_body)


def _pool_body(batch_hbm, za_hbm, zb_hbm, pout, zrow, bbuf, acc, sem):
    c = lax.axis_index("c")
    s = lax.axis_index("s")
    rpt = NGRAPH // NS
    rows = N // (NC * NS)

    for fh in range(2):
        z_hbm = za_hbm if fh == 0 else zb_hbm

        _zero_rows16(zrow, rpt)
        pltpu.sync_copy(zrow.at[pl.ds(0, rpt)], acc.at[pl.ds(s * rpt, rpt)])
        plsc.subcore_barrier()

        def chunk(ch, _):
            base = (c * NS + s) * rows + ch * 496
            pltpu.sync_copy(batch_hbm.at[pl.ds(base, 496)], bbuf)
            pltpu.sync_copy(z_hbm.at[pl.ds(base, 496)], zrow)
            pltpu.sync_copy(zrow, acc.at[bbuf], add=True)
            return 0

        lax.fori_loop(0, rows // 496, chunk, 0)
        plsc.subcore_barrier()

        pltpu.sync_copy(acc.at[pl.ds(s * rpt, rpt)], zrow.at[pl.ds(0, rpt)])
        pltpu.sync_copy(zrow.at[pl.ds(0, rpt)],
                        pout.at[pl.ds((c * 2 + fh) * NGRAPH + s * rpt, rpt)])
        plsc.subcore_barrier()


_pool_call = functools.partial(
    pl.kernel,
    out_type=jax.ShapeDtypeStruct((NC * 2 * NGRAPH, FH), jnp.float32),
    mesh=_MESH,
    compiler_params=_SC_PARAMS,
    scratch_types=[
        pltpu.VMEM((496, FH), jnp.float32),
        pltpu.VMEM((496,), jnp.int32),
        pltpu.VMEM_SHARED((NGRAPH, FH), jnp.float32),
        pltpu.SemaphoreType.DMA,
    ],
)(_pool_body)


OBLK = 512
NOBLK = NOCT // OBLK


def _dis_from(degrep):
    deg = degrep + 1.0
    return jnp.where(deg > 0, lax.rsqrt(deg), 0.0)


def _pro_body(x3_ref, wt_ref, dg_ref, ua_ref, ub_ref):
    wt = wt_ref[...]
    ys = [jnp.dot(x3_ref[:, j, :], wt, preferred_element_type=jnp.float32)
          for j in range(8)]
    ya = jnp.concatenate([y[:, :FH] for y in ys], axis=1)
    yb = jnp.concatenate([y[:, FH:] for y in ys], axis=1)
    dis = _dis_from(dg_ref[...])
    ua_ref[...] = ya * dis
    ub_ref[...] = yb * dis


def _prologue(x3, wt, degrep):
    o16 = pl.BlockSpec((OBLK, 128), lambda i: (i, 0))
    return pl.pallas_call(
        _pro_body,
        grid=(NOBLK,),
        in_specs=[
            pl.BlockSpec((OBLK, 8, 128), lambda i: (i, 0, 0)),
            pl.BlockSpec((DIN, DP), lambda i: (0, 0)),
            o16,
        ],
        out_specs=[o16, o16],
        out_shape=[
            jax.ShapeDtypeStruct((NOCT, 128), jnp.float32),
            jax.ShapeDtypeStruct((NOCT, 128), jnp.float32),
        ],
    )(x3, wt, degrep)


def _mid_body(ea_ref, eb_ref, ua_ref, ub_ref, dg_ref, wa_ref, wb_ref):
    deg = dg_ref[...] + 1.0
    d2 = jnp.where(deg > 0, 1.0 / deg, 0.0)
    wa_ref[...] = d2 * (ea_ref[...] + ua_ref[...])
    wb_ref[...] = d2 * (eb_ref[...] + ub_ref[...])


def _mid(ea, eb, ua, ub, degrep):
    o16 = pl.BlockSpec((OBLK, 128), lambda i: (i, 0))
    return pl.pallas_call(
        _mid_body,
        grid=(NOBLK,),
        in_specs=[o16, o16, o16, o16, o16],
        out_specs=[o16, o16],
        out_shape=[jax.ShapeDtypeStruct((NOCT, 128), jnp.float32),
                   jax.ShapeDtypeStruct((NOCT, 128), jnp.float32)],
    )(ea, eb, ua, ub, degrep)


def _post_body(ea_ref, eb_ref, wa_ref, wb_ref, dg_ref, ba_ref, bb_ref,
               gs_ref, r8_ref, za_ref, zb_ref):
    dis = _dis_from(dg_ref[...])
    ha = dis * (ea_ref[...] + wa_ref[...]) + ba_ref[...]
    hb = dis * (eb_ref[...] + wb_ref[...]) + bb_ref[...]
    exa = jnp.exp(jnp.minimum(ha, 85.0))
    exb = jnp.exp(jnp.minimum(hb, 85.0))
    s8 = (jnp.dot(exa, gs_ref[...], preferred_element_type=jnp.float32)
          + jnp.dot(exb, gs_ref[...], preferred_element_type=jnp.float32))
    rep = jnp.dot(1.0 / s8, r8_ref[...], preferred_element_type=jnp.float32)
    za_ref[...] = exa * rep
    zb_ref[...] = exb * rep


def _post(ea, eb, wa, wb, degrep, ba, bb, gsum, r8):
    o16 = pl.BlockSpec((OBLK, 128), lambda i: (i, 0))
    return pl.pallas_call(
        _post_body,
        grid=(NOBLK,),
        in_specs=[o16, o16, o16, o16, o16,
                  pl.BlockSpec((1, 128), lambda i: (0, 0)),
                  pl.BlockSpec((1, 128), lambda i: (0, 0)),
                  pl.BlockSpec((128, 8), lambda i: (0, 0)),
                  pl.BlockSpec((8, 128), lambda i: (0, 0))],
        out_specs=[o16, o16],
        out_shape=[jax.ShapeDtypeStruct((NOCT, 128), jnp.float32),
                   jax.ShapeDtypeStruct((NOCT, 128), jnp.float32)],
    )(ea, eb, wa, wb, degrep, ba, bb, gsum, r8)


def _fin_body(p0a_ref, p1a_ref, p0b_ref, p1b_ref, w_ref, b_ref, o_ref):
    p = jnp.concatenate([p0a_ref[...] + p1a_ref[...],
                         p0b_ref[...] + p1b_ref[...]], axis=1)
    lg = jnp.dot(p, w_ref[...], preferred_element_type=jnp.float32) + b_ref[...]
    m = jnp.max(lg, axis=-1, keepdims=True)
    ex = jnp.exp(lg - m)
    o_ref[...] = ex / jnp.sum(ex, axis=-1, keepdims=True)


def _final(p0a, p1a, p0b, p1b, fcwt, fcb):
    return pl.pallas_call(
        _fin_body,
        out_shape=jax.ShapeDtypeStruct((NGRAPH, 8), jnp.float32),
    )(p0a, p1a, p0b, p1b, fcwt, fcb)


def kernel(len_y, x, index, batch, weight, lin_W, lin_b, fc_W, fc_b):
    del len_y
    row = index[0]
    col = index[1]
    f32 = jnp.float32

    wrep = jnp.broadcast_to(jnp.tile(weight, L)[:, None], (2048, FH))
    w16 = jnp.repeat(weight, L)
    zer = jnp.zeros((ACC_ROWS // NS, FH), f32)

    degrep16 = _deg_call(col, wrep, zer)
    degrep = degrep16.reshape(NOCT, 128)

    wt = jnp.zeros((DP, DIN), f32).at[:HID].set(lin_W).T
    x3 = x.reshape(NOCT, 8, DIN)
    ua_o, ub_o = _prologue(x3, wt, degrep)

    def flat(a):
        return a.reshape(N, FH)

    def oct_(a):
        return a.reshape(NOCT, 128)

    e1a, e1b = _hop_call(row, col, w16, zer, flat(ua_o), flat(ub_o))
    wa_o, wb_o = _mid(oct_(e1a), oct_(e1b), ua_o, ub_o, degrep)
    e2a, e2b = _hop_call(row, col, w16, zer, flat(wa_o), flat(wb_o))

    lane = jnp.arange(128)
    ba = jnp.tile(lin_b[:FH], 8)
    bbpat = jnp.concatenate([lin_b[FH:HID], jnp.full((2,), -1e30, f32)])
    bb = jnp.tile(bbpat, 8)
    gsum = (lane[:, None] // FH == jnp.arange(8)[None, :]).astype(f32)
    r8 = gsum.T

    za_o, zb_o = _post(oct_(e2a), oct_(e2b), wa_o, wb_o, degrep,
                       ba.reshape(1, 128), bb.reshape(1, 128), gsum, r8)

    pools = _pool_call(batch, flat(za_o), flat(zb_o))
    p0a = pools[:NGRAPH]
    p0b = pools[NGRAPH:2 * NGRAPH]
    p1a = pools[2 * NGRAPH:3 * NGRAPH]
    p1b = pools[3 * NGRAPH:]

    fcw = jnp.zeros((8, DP), f32).at[:NCLS, :HID].set(fc_W)
    fcb = jnp.concatenate([fc_b, jnp.full((5,), -1e30, f32)])
    out8 = _final(p0a, p1a, p0b, p1b, fcw.T, fcb.reshape(1, 8))
    return out8[:, :NCLS]


# speedup vs baseline: 1.3320x; 1.0446x over previous
import functools

import jax
import jax.numpy as jnp
from jax import lax
from jax.experimental import pallas as pl
from jax.experimental.pallas import tpu as pltpu
from jax.experimental.pallas import tpu_sc as plsc

N = 126976
E = 262144
DIN = 128
HID = 30
DP = 32
FH = 16
NGRAPH = 2048
NCLS = 3
WP = 128

NC, NS, L = 2, 16, 16
HALF = N // 2
DUMP = 1024
ACC_ROWS = HALF + DUMP
EPT_M = E // NS
CH = 256
NCHUNK = EPT_M // CH
NOCT = N // 8

_MESH = plsc.VectorSubcoreMesh(core_axis_name="c", subcore_axis_name="s")
_SC_PARAMS = pltpu.CompilerParams(use_tc_tiling_on_sc=False)


def _zero_rows16(ref, nrows):
    zv = jnp.zeros((L,), jnp.float32)

    def body(i, _):
        ref[i, pl.ds(0, L)] = zv
        return 0

    lax.fori_loop(0, nrows, body, 0)


def _zero_acc(acc, s, zeros_hbm):
    rows_per_tile = ACC_ROWS // NS
    pltpu.sync_copy(zeros_hbm, acc.at[pl.ds(s * rows_per_tile, rows_per_tile)])


def _write_half(acc, eout, c, s, bounce):
    del bounce
    rpt = HALF // NS
    o = s * rpt
    pltpu.sync_copy(acc.at[pl.ds(o, rpt)], eout.at[pl.ds(c * HALF + o, rpt)])


DSTROWS = 2048 // CH


def _deg_body(col_hbm, wrep_hbm, zer_hbm, degrep, dstl, wrep, colbuf, dstbuf,
              dstbuf2, zbuf, acc, sem):
    c = lax.axis_index("c")
    s = lax.axis_index("s")
    lo = c * HALF

    pltpu.sync_copy(wrep_hbm, wrep)
    _zero_acc(acc, s, zer_hbm)
    plsc.subcore_barrier()

    def chunk(ch, _):
        ebase = s * EPT_M + ch * 2048
        pltpu.sync_copy(col_hbm.at[pl.ds(ebase, 2048)], colbuf)

        @plsc.parallel_loop(0, 2048 // L, unroll=2)
        def _(g):
            col_v = colbuf[pl.ds(g * L, L)]
            inhalf = (col_v >= lo) & (col_v < lo + HALF)
            dst = jnp.where(inhalf, col_v - lo, HALF + (col_v & (DUMP - 1)))
            dstbuf[pl.ds(g * L, L)] = dst
            dstbuf2[g // (CH // L), pl.ds((g % (CH // L)) * L, L)] = dst

        pltpu.sync_copy(wrep, acc.at[dstbuf], add=True)
        rowbase = (c * NS + s) * NCHUNK + ch * DSTROWS
        pltpu.sync_copy(dstbuf2, dstl.at[pl.ds(rowbase, DSTROWS)])
        return 0

    lax.fori_loop(0, EPT_M // 2048, chunk, 0)
    plsc.subcore_barrier()
    _write_half(acc, degrep, c, s, zbuf)


_deg_call = functools.partial(
    pl.kernel,
    out_type=[jax.ShapeDtypeStruct((N, FH), jnp.float32),
              jax.ShapeDtypeStruct((NC * NS * NCHUNK, CH), jnp.int32)],
    mesh=_MESH,
    compiler_params=_SC_PARAMS,
    scratch_types=[
        pltpu.VMEM((2048, FH), jnp.float32),
        pltpu.VMEM((2048,), jnp.int32),
        pltpu.VMEM((2048,), jnp.int32),
        pltpu.VMEM((DSTROWS, CH), jnp.int32),
        pltpu.VMEM((8, FH), jnp.float32),
        pltpu.VMEM_SHARED((ACC_ROWS, FH), jnp.float32),
        pltpu.SemaphoreType.DMA,
    ],
)(_deg_body)


def _hop_body(row_hbm, dstl_hbm, w16_hbm, zer_hbm, ga_hbm, gb_hbm,
              eout_a, eout_b,
              w16, rowall, dstall, gb0, gb1, gb2, gb3, acc,
              gs0, gs1, gs2, gs3, ss0, ss1, ss2, ss3):
    c = lax.axis_index("c")
    s = lax.axis_index("s")
    gbufs = (gb0, gb1, gb2, gb3)
    gsems = (gs0, gs1, gs2, gs3)
    ssems = (ss0, ss1, ss2, ss3)

    pltpu.sync_copy(w16_hbm, w16)
    ebase = s * EPT_M
    pltpu.sync_copy(row_hbm.at[pl.ds(ebase, EPT_M)], rowall)
    pltpu.sync_copy(dstl_hbm.at[pl.ds((c * NS + s) * NCHUNK, NCHUNK)], dstall)

    for fh in range(2):
        g_hbm = ga_hbm if fh == 0 else gb_hbm
        eout = eout_a if fh == 0 else eout_b

        _zero_acc(acc, s, zer_hbm)
        plsc.subcore_barrier()

        def start_gather(chv, b):
            pltpu.async_copy(g_hbm.at[rowall.at[pl.ds(chv * CH, CH)]],
                             gbufs[b], gsems[b])

        def wait_slot(sems, b):
            pltpu.make_async_copy(g_hbm.at[pl.ds(0, CH)], gbufs[b],
                                  sems[b]).wait()

        def start_scatter(chv, b):
            pltpu.async_copy(gbufs[b], acc.at[dstall.at[chv]],
                             ssems[b], add=True)

        def scale(gbuf):
            @plsc.parallel_loop(0, CH // L, unroll=2)
            def _(g):
                off = g * L
                wb = (g & 7) * (L * L)
                for k in range(L):
                    gbuf[off + k, pl.ds(0, L)] = (
                        gbuf[off + k, pl.ds(0, L)] * w16[pl.ds(wb + k * L, L)])

        start_gather(0, 0)
        start_gather(1, 1)

        def quad(q, _):
            for j in range(4):
                chv = q * 4 + j
                b2 = (j + 2) % 4
                wait_slot(gsems, j)
                scale(gbufs[j])
                start_scatter(chv, j)
                if j < 2:
                    @pl.when(q >= 1)
                    def _():
                        wait_slot(ssems, b2)
                    start_gather(chv + 2, b2)
                else:
                    wait_slot(ssems, b2)

                    @pl.when(q < NCHUNK // 4 - 1)
                    def _():
                        start_gather(chv + 2, b2)
            return 0

        lax.fori_loop(0, NCHUNK // 4, quad, 0)
        wait_slot(ssems, 2)
        wait_slot(ssems, 3)
        plsc.subcore_barrier()

        _write_half(acc, eout, c, s, gb0)
        plsc.subcore_barrier()


_hop_call = functools.partial(
    pl.kernel,
    out_type=[jax.ShapeDtypeStruct((N, FH), jnp.float32),
              jax.ShapeDtypeStruct((N, FH), jnp.float32)],
    mesh=_MESH,
    compiler_params=_SC_PARAMS,
    scratch_types=[
        pltpu.VMEM((WP * L,), jnp.float32),
        pltpu.VMEM((EPT_M,), jnp.int32),
        pltpu.VMEM((NCHUNK, CH), jnp.int32),
        pltpu.VMEM((CH, FH), jnp.float32),
        pltpu.VMEM((CH, FH), jnp.float32),
        pltpu.VMEM((CH, FH), jnp.float32),
        pltpu.VMEM((CH, FH), jnp.float32),
        pltpu.VMEM_SHARED((ACC_ROWS, FH), jnp.float32),
        pltpu.SemaphoreType.DMA,
        pltpu.SemaphoreType.DMA,
        pltpu.SemaphoreType.DMA,
        pltpu.SemaphoreType.DMA,
        pltpu.SemaphoreType.DMA,
        pltpu.SemaphoreType.DMA,
        pltpu.SemaphoreType.DMA,
        pltpu.SemaphoreType.DMA,
    ],
)(_hop_body)


def _pool_body(batch_hbm, za_hbm, zb_hbm, pout, zrow, bbuf, acc, sem):
    c = lax.axis_index("c")
    s = lax.axis_index("s")
    rpt = NGRAPH // NS
    rows = N // (NC * NS)

    for fh in range(2):
        z_hbm = za_hbm if fh == 0 else zb_hbm

        _zero_rows16(zrow, rpt)
        pltpu.sync_copy(zrow.at[pl.ds(0, rpt)], acc.at[pl.ds(s * rpt, rpt)])
        plsc.subcore_barrier()

        def chunk(ch, _):
            base = (c * NS + s) * rows + ch * 496
            pltpu.sync_copy(batch_hbm.at[pl.ds(base, 496)], bbuf)
            pltpu.sync_copy(z_hbm.at[pl.ds(base, 496)], zrow)
            pltpu.sync_copy(zrow, acc.at[bbuf], add=True)
            return 0

        lax.fori_loop(0, rows // 496, chunk, 0)
        plsc.subcore_barrier()

        pltpu.sync_copy(acc.at[pl.ds(s * rpt, rpt)], zrow.at[pl.ds(0, rpt)])
        pltpu.sync_copy(zrow.at[pl.ds(0, rpt)],
                        pout.at[pl.ds((c * 2 + fh) * NGRAPH + s * rpt, rpt)])
        plsc.subcore_barrier()


_pool_call = functools.partial(
    pl.kernel,
    out_type=jax.ShapeDtypeStruct((NC * 2 * NGRAPH, FH), jnp.float32),
    mesh=_MESH,
    compiler_params=_SC_PARAMS,
    scratch_types=[
        pltpu.VMEM((496, FH), jnp.float32),
        pltpu.VMEM((496,), jnp.int32),
        pltpu.VMEM_SHARED((NGRAPH, FH), jnp.float32),
        pltpu.SemaphoreType.DMA,
    ],
)(_pool_body)


OBLK = 512
NOBLK = NOCT // OBLK


def _dis_from(degrep):
    deg = degrep + 1.0
    return jnp.where(deg > 0, lax.rsqrt(deg), 0.0)


def _pro_body(x3_ref, wt_ref, dg_ref, ua_ref, ub_ref):
    wt = wt_ref[...]
    ys = [jnp.dot(x3_ref[:, j, :], wt, preferred_element_type=jnp.float32)
          for j in range(8)]
    ya = jnp.concatenate([y[:, :FH] for y in ys], axis=1)
    yb = jnp.concatenate([y[:, FH:] for y in ys], axis=1)
    dis = _dis_from(dg_ref[...])
    ua_ref[...] = ya * dis
    ub_ref[...] = yb * dis


def _prologue(x3, wt, degrep):
    o16 = pl.BlockSpec((OBLK, 128), lambda i: (i, 0))
    return pl.pallas_call(
        _pro_body,
        grid=(NOBLK,),
        in_specs=[
            pl.BlockSpec((OBLK, 8, 128), lambda i: (i, 0, 0)),
            pl.BlockSpec((DIN, DP), lambda i: (0, 0)),
            o16,
        ],
        out_specs=[o16, o16],
        out_shape=[
            jax.ShapeDtypeStruct((NOCT, 128), jnp.float32),
            jax.ShapeDtypeStruct((NOCT, 128), jnp.float32),
        ],
    )(x3, wt, degrep)


def _mid_body(ea_ref, eb_ref, ua_ref, ub_ref, dg_ref, wa_ref, wb_ref):
    deg = dg_ref[...] + 1.0
    d2 = jnp.where(deg > 0, 1.0 / deg, 0.0)
    wa_ref[...] = d2 * (ea_ref[...] + ua_ref[...])
    wb_ref[...] = d2 * (eb_ref[...] + ub_ref[...])


def _mid(ea, eb, ua, ub, degrep):
    o16 = pl.BlockSpec((OBLK, 128), lambda i: (i, 0))
    return pl.pallas_call(
        _mid_body,
        grid=(NOBLK,),
        in_specs=[o16, o16, o16, o16, o16],
        out_specs=[o16, o16],
        out_shape=[jax.ShapeDtypeStruct((NOCT, 128), jnp.float32),
                   jax.ShapeDtypeStruct((NOCT, 128), jnp.float32)],
    )(ea, eb, ua, ub, degrep)


def _post_body(ea_ref, eb_ref, wa_ref, wb_ref, dg_ref, ba_ref, bb_ref,
               gs_ref, r8_ref, za_ref, zb_ref):
    dis = _dis_from(dg_ref[...])
    ha = dis * (ea_ref[...] + wa_ref[...]) + ba_ref[...]
    hb = dis * (eb_ref[...] + wb_ref[...]) + bb_ref[...]
    exa = jnp.exp(jnp.minimum(ha, 85.0))
    exb = jnp.exp(jnp.minimum(hb, 85.0))
    s8 = (jnp.dot(exa, gs_ref[...], preferred_element_type=jnp.float32)
          + jnp.dot(exb, gs_ref[...], preferred_element_type=jnp.float32))
    rep = jnp.dot(1.0 / s8, r8_ref[...], preferred_element_type=jnp.float32)
    za_ref[...] = exa * rep
    zb_ref[...] = exb * rep


def _post(ea, eb, wa, wb, degrep, ba, bb, gsum, r8):
    o16 = pl.BlockSpec((OBLK, 128), lambda i: (i, 0))
    return pl.pallas_call(
        _post_body,
        grid=(NOBLK,),
        in_specs=[o16, o16, o16, o16, o16,
                  pl.BlockSpec((1, 128), lambda i: (0, 0)),
                  pl.BlockSpec((1, 128), lambda i: (0, 0)),
                  pl.BlockSpec((128, 8), lambda i: (0, 0)),
                  pl.BlockSpec((8, 128), lambda i: (0, 0))],
        out_specs=[o16, o16],
        out_shape=[jax.ShapeDtypeStruct((NOCT, 128), jnp.float32),
                   jax.ShapeDtypeStruct((NOCT, 128), jnp.float32)],
    )(ea, eb, wa, wb, degrep, ba, bb, gsum, r8)


def _fin_body(p0a_ref, p1a_ref, p0b_ref, p1b_ref, w_ref, b_ref, o_ref):
    p = jnp.concatenate([p0a_ref[...] + p1a_ref[...],
                         p0b_ref[...] + p1b_ref[...]], axis=1)
    lg = jnp.dot(p, w_ref[...], preferred_element_type=jnp.float32) + b_ref[...]
    m = jnp.max(lg, axis=-1, keepdims=True)
    ex = jnp.exp(lg - m)
    o_ref[...] = ex / jnp.sum(ex, axis=-1, keepdims=True)


def _final(p0a, p1a, p0b, p1b, fcwt, fcb):
    return pl.pallas_call(
        _fin_body,
        out_shape=jax.ShapeDtypeStruct((NGRAPH, 8), jnp.float32),
    )(p0a, p1a, p0b, p1b, fcwt, fcb)


def kernel(len_y, x, index, batch, weight, lin_W, lin_b, fc_W, fc_b):
    del len_y
    row = index[0]
    col = index[1]
    f32 = jnp.float32

    wrep = jnp.broadcast_to(jnp.tile(weight, L)[:, None], (2048, FH))
    w16 = jnp.repeat(weight, L)
    zer = jnp.zeros((ACC_ROWS // NS, FH), f32)

    degrep16, dstl = _deg_call(col, wrep, zer)
    degrep = degrep16.reshape(NOCT, 128)

    wt = jnp.zeros((DP, DIN), f32).at[:HID].set(lin_W).T
    x3 = x.reshape(NOCT, 8, DIN)
    ua_o, ub_o = _prologue(x3, wt, degrep)

    def flat(a):
        return a.reshape(N, FH)

    def oct_(a):
        return a.reshape(NOCT, 128)

    e1a, e1b = _hop_call(row, dstl, w16, zer, flat(ua_o), flat(ub_o))
    wa_o, wb_o = _mid(oct_(e1a), oct_(e1b), ua_o, ub_o, degrep)
    e2a, e2b = _hop_call(row, dstl, w16, zer, flat(wa_o), flat(wb_o))

    lane = jnp.arange(128)
    ba = jnp.tile(lin_b[:FH], 8)
    bbpat = jnp.concatenate([lin_b[FH:HID], jnp.full((2,), -1e30, f32)])
    bb = jnp.tile(bbpat, 8)
    gsum = (lane[:, None] // FH == jnp.arange(8)[None, :]).astype(f32)
    r8 = gsum.T

    za_o, zb_o = _post(oct_(e2a), oct_(e2b), wa_o, wb_o, degrep,
                       ba.reshape(1, 128), bb.reshape(1, 128), gsum, r8)

    pools = _pool_call(batch, flat(za_o), flat(zb_o))
    p0a = pools[:NGRAPH]
    p0b = pools[NGRAPH:2 * NGRAPH]
    p1a = pools[2 * NGRAPH:3 * NGRAPH]
    p1b = pools[3 * NGRAPH:]

    fcw = jnp.zeros((8, DP), f32).at[:NCLS, :HID].set(fc_W)
    fcb = jnp.concatenate([fc_b, jnp.full((5,), -1e30, f32)])
    out8 = _final(p0a, p1a, p0b, p1b, fcw.T, fcb.reshape(1, 8))
    return out8[:, :NCLS]


# speedup vs baseline: 1.3440x; 1.0090x over previous
import functools

import jax
import jax.numpy as jnp
from jax import lax
from jax.experimental import pallas as pl
from jax.experimental.pallas import tpu as pltpu
from jax.experimental.pallas import tpu_sc as plsc

N = 126976
E = 262144
DIN = 128
HID = 30
DP = 32
FH = 16
NGRAPH = 2048
NCLS = 3
WP = 128

NC, NS, L = 2, 16, 16
HALF = N // 2
DUMP = 1024
ACC_ROWS = HALF + DUMP
EPT_M = E // NS
CH = 256
NCHUNK = EPT_M // CH
NOCT = N // 8

_MESH = plsc.VectorSubcoreMesh(core_axis_name="c", subcore_axis_name="s")
_SC_PARAMS = pltpu.CompilerParams(use_tc_tiling_on_sc=False)


def _zero_rows16(ref, nrows):
    zv = jnp.zeros((L,), jnp.float32)

    def body(i, _):
        ref[i, pl.ds(0, L)] = zv
        return 0

    lax.fori_loop(0, nrows, body, 0)


def _zero_acc(acc, s, zeros_hbm):
    rows_per_tile = ACC_ROWS // NS
    pltpu.sync_copy(zeros_hbm, acc.at[pl.ds(s * rows_per_tile, rows_per_tile)])


def _write_half(acc, eout, c, s, bounce):
    del bounce
    rpt = HALF // NS
    o = s * rpt
    pltpu.sync_copy(acc.at[pl.ds(o, rpt)], eout.at[pl.ds(c * HALF + o, rpt)])


DSTROWS = 2048 // CH


def _deg_body(col_hbm, wrep_hbm, zer_hbm, degrep, dstl, wrep, colbuf, dstbuf,
              dstbuf2, zbuf, acc, sem):
    c = lax.axis_index("c")
    s = lax.axis_index("s")
    lo = c * HALF

    pltpu.sync_copy(wrep_hbm, wrep)
    _zero_acc(acc, s, zer_hbm)
    plsc.subcore_barrier()

    def chunk(ch, _):
        ebase = s * EPT_M + ch * 2048
        pltpu.sync_copy(col_hbm.at[pl.ds(ebase, 2048)], colbuf)

        @plsc.parallel_loop(0, 2048 // L, unroll=2)
        def _(g):
            col_v = colbuf[pl.ds(g * L, L)]
            inhalf = (col_v >= lo) & (col_v < lo + HALF)
            dst = jnp.where(inhalf, col_v - lo, HALF + (col_v & (DUMP - 1)))
            dstbuf[pl.ds(g * L, L)] = dst
            dstbuf2[g // (CH // L), pl.ds((g % (CH // L)) * L, L)] = dst

        pltpu.sync_copy(wrep, acc.at[dstbuf], add=True)
        rowbase = (c * NS + s) * NCHUNK + ch * DSTROWS
        pltpu.sync_copy(dstbuf2, dstl.at[pl.ds(rowbase, DSTROWS)])
        return 0

    lax.fori_loop(0, EPT_M // 2048, chunk, 0)
    plsc.subcore_barrier()
    _write_half(acc, degrep, c, s, zbuf)


_deg_call = functools.partial(
    pl.kernel,
    out_type=[jax.ShapeDtypeStruct((N, FH), jnp.float32),
              jax.ShapeDtypeStruct((NC * NS * NCHUNK, CH), jnp.int32)],
    mesh=_MESH,
    compiler_params=_SC_PARAMS,
    scratch_types=[
        pltpu.VMEM((2048, FH), jnp.float32),
        pltpu.VMEM((2048,), jnp.int32),
        pltpu.VMEM((2048,), jnp.int32),
        pltpu.VMEM((DSTROWS, CH), jnp.int32),
        pltpu.VMEM((8, FH), jnp.float32),
        pltpu.VMEM_SHARED((ACC_ROWS, FH), jnp.float32),
        pltpu.SemaphoreType.DMA,
    ],
)(_deg_body)


def _hop_body(row_hbm, dstl_hbm, w16_hbm, zer_hbm, ga_hbm, gb_hbm,
              eout_a, eout_b,
              w16, rowall, dstall, gb0, gb1, gb2, gb3, acc,
              gs0, gs1, gs2, gs3, ss0, ss1, ss2, ss3):
    c = lax.axis_index("c")
    s = lax.axis_index("s")
    gbufs = (gb0, gb1, gb2, gb3)
    gsems = (gs0, gs1, gs2, gs3)
    ssems = (ss0, ss1, ss2, ss3)

    pltpu.sync_copy(w16_hbm, w16)
    ebase = s * EPT_M
    pltpu.sync_copy(row_hbm.at[pl.ds(ebase, EPT_M)], rowall)
    pltpu.sync_copy(dstl_hbm.at[pl.ds((c * NS + s) * NCHUNK, NCHUNK)], dstall)

    for fh in range(2):
        g_hbm = ga_hbm if fh == 0 else gb_hbm
        eout = eout_a if fh == 0 else eout_b

        _zero_acc(acc, s, zer_hbm)
        plsc.subcore_barrier()

        def start_gather(chv, b):
            pltpu.async_copy(g_hbm.at[rowall.at[pl.ds(chv * CH, CH)]],
                             gbufs[b], gsems[b])

        def wait_slot(sems, b):
            pltpu.make_async_copy(g_hbm.at[pl.ds(0, CH)], gbufs[b],
                                  sems[b]).wait()

        def start_scatter(chv, b):
            pltpu.async_copy(gbufs[b], acc.at[dstall.at[chv]],
                             ssems[b], add=True)

        def scale(gbuf):
            @plsc.parallel_loop(0, CH // L, unroll=4)
            def _(g):
                off = g * L
                wb = (g & 7) * (L * L)
                for k in range(L):
                    gbuf[off + k, pl.ds(0, L)] = (
                        gbuf[off + k, pl.ds(0, L)] * w16[pl.ds(wb + k * L, L)])

        start_gather(0, 0)
        start_gather(1, 1)

        def quad(q, _):
            for j in range(4):
                chv = q * 4 + j
                b2 = (j + 2) % 4
                wait_slot(gsems, j)
                scale(gbufs[j])
                start_scatter(chv, j)
                if j < 2:
                    @pl.when(q >= 1)
                    def _():
                        wait_slot(ssems, b2)
                    start_gather(chv + 2, b2)
                else:
                    wait_slot(ssems, b2)

                    @pl.when(q < NCHUNK // 4 - 1)
                    def _():
                        start_gather(chv + 2, b2)
            return 0

        lax.fori_loop(0, NCHUNK // 4, quad, 0)
        wait_slot(ssems, 2)
        wait_slot(ssems, 3)
        plsc.subcore_barrier()

        _write_half(acc, eout, c, s, gb0)
        plsc.subcore_barrier()


_hop_call = functools.partial(
    pl.kernel,
    out_type=[jax.ShapeDtypeStruct((N, FH), jnp.float32),
              jax.ShapeDtypeStruct((N, FH), jnp.float32)],
    mesh=_MESH,
    compiler_params=_SC_PARAMS,
    scratch_types=[
        pltpu.VMEM((WP * L,), jnp.float32),
        pltpu.VMEM((EPT_M,), jnp.int32),
        pltpu.VMEM((NCHUNK, CH), jnp.int32),
        pltpu.VMEM((CH, FH), jnp.float32),
        pltpu.VMEM((CH, FH), jnp.float32),
        pltpu.VMEM((CH, FH), jnp.float32),
        pltpu.VMEM((CH, FH), jnp.float32),
        pltpu.VMEM_SHARED((ACC_ROWS, FH), jnp.float32),
        pltpu.SemaphoreType.DMA,
        pltpu.SemaphoreType.DMA,
        pltpu.SemaphoreType.DMA,
        pltpu.SemaphoreType.DMA,
        pltpu.SemaphoreType.DMA,
        pltpu.SemaphoreType.DMA,
        pltpu.SemaphoreType.DMA,
        pltpu.SemaphoreType.DMA,
    ],
)(_hop_body)


def _pool_body(batch_hbm, za_hbm, zb_hbm, pout, zrow, bbuf, acc, sem):
    c = lax.axis_index("c")
    s = lax.axis_index("s")
    rpt = NGRAPH // NS
    rows = N // (NC * NS)

    for fh in range(2):
        z_hbm = za_hbm if fh == 0 else zb_hbm

        _zero_rows16(zrow, rpt)
        pltpu.sync_copy(zrow.at[pl.ds(0, rpt)], acc.at[pl.ds(s * rpt, rpt)])
        plsc.subcore_barrier()

        def chunk(ch, _):
            base = (c * NS + s) * rows + ch * 496
            pltpu.sync_copy(batch_hbm.at[pl.ds(base, 496)], bbuf)
            pltpu.sync_copy(z_hbm.at[pl.ds(base, 496)], zrow)
            pltpu.sync_copy(zrow, acc.at[bbuf], add=True)
            return 0

        lax.fori_loop(0, rows // 496, chunk, 0)
        plsc.subcore_barrier()

        pltpu.sync_copy(acc.at[pl.ds(s * rpt, rpt)], zrow.at[pl.ds(0, rpt)])
        pltpu.sync_copy(zrow.at[pl.ds(0, rpt)],
                        pout.at[pl.ds((c * 2 + fh) * NGRAPH + s * rpt, rpt)])
        plsc.subcore_barrier()


_pool_call = functools.partial(
    pl.kernel,
    out_type=jax.ShapeDtypeStruct((NC * 2 * NGRAPH, FH), jnp.float32),
    mesh=_MESH,
    compiler_params=_SC_PARAMS,
    scratch_types=[
        pltpu.VMEM((496, FH), jnp.float32),
        pltpu.VMEM((496,), jnp.int32),
        pltpu.VMEM_SHARED((NGRAPH, FH), jnp.float32),
        pltpu.SemaphoreType.DMA,
    ],
)(_pool_body)


OBLK = 512
NOBLK = NOCT // OBLK


def _dis_from(degrep):
    deg = degrep + 1.0
    return jnp.where(deg > 0, lax.rsqrt(deg), 0.0)


def _pro_body(x3_ref, wt_ref, dg_ref, ua_ref, ub_ref):
    wt = wt_ref[...]
    ys = [jnp.dot(x3_ref[:, j, :], wt, preferred_element_type=jnp.float32)
          for j in range(8)]
    ya = jnp.concatenate([y[:, :FH] for y in ys], axis=1)
    yb = jnp.concatenate([y[:, FH:] for y in ys], axis=1)
    dis = _dis_from(dg_ref[...])
    ua_ref[...] = ya * dis
    ub_ref[...] = yb * dis


def _prologue(x3, wt, degrep):
    o16 = pl.BlockSpec((OBLK, 128), lambda i: (i, 0))
    return pl.pallas_call(
        _pro_body,
        grid=(NOBLK,),
        in_specs=[
            pl.BlockSpec((OBLK, 8, 128), lambda i: (i, 0, 0)),
            pl.BlockSpec((DIN, DP), lambda i: (0, 0)),
            o16,
        ],
        out_specs=[o16, o16],
        out_shape=[
            jax.ShapeDtypeStruct((NOCT, 128), jnp.float32),
            jax.ShapeDtypeStruct((NOCT, 128), jnp.float32),
        ],
    )(x3, wt, degrep)


def _mid_body(ea_ref, eb_ref, ua_ref, ub_ref, dg_ref, wa_ref, wb_ref):
    deg = dg_ref[...] + 1.0
    d2 = jnp.where(deg > 0, 1.0 / deg, 0.0)
    wa_ref[...] = d2 * (ea_ref[...] + ua_ref[...])
    wb_ref[...] = d2 * (eb_ref[...] + ub_ref[...])


def _mid(ea, eb, ua, ub, degrep):
    o16 = pl.BlockSpec((OBLK, 128), lambda i: (i, 0))
    return pl.pallas_call(
        _mid_body,
        grid=(NOBLK,),
        in_specs=[o16, o16, o16, o16, o16],
        out_specs=[o16, o16],
        out_shape=[jax.ShapeDtypeStruct((NOCT, 128), jnp.float32),
                   jax.ShapeDtypeStruct((NOCT, 128), jnp.float32)],
    )(ea, eb, ua, ub, degrep)


def _post_body(ea_ref, eb_ref, wa_ref, wb_ref, dg_ref, ba_ref, bb_ref,
               gs_ref, r8_ref, za_ref, zb_ref):
    dis = _dis_from(dg_ref[...])
    ha = dis * (ea_ref[...] + wa_ref[...]) + ba_ref[...]
    hb = dis * (eb_ref[...] + wb_ref[...]) + bb_ref[...]
    exa = jnp.exp(jnp.minimum(ha, 85.0))
    exb = jnp.exp(jnp.minimum(hb, 85.0))
    s8 = (jnp.dot(exa, gs_ref[...], preferred_element_type=jnp.float32)
          + jnp.dot(exb, gs_ref[...], preferred_element_type=jnp.float32))
    rep = jnp.dot(1.0 / s8, r8_ref[...], preferred_element_type=jnp.float32)
    za_ref[...] = exa * rep
    zb_ref[...] = exb * rep


def _post(ea, eb, wa, wb, degrep, ba, bb, gsum, r8):
    o16 = pl.BlockSpec((OBLK, 128), lambda i: (i, 0))
    return pl.pallas_call(
        _post_body,
        grid=(NOBLK,),
        in_specs=[o16, o16, o16, o16, o16,
                  pl.BlockSpec((1, 128), lambda i: (0, 0)),
                  pl.BlockSpec((1, 128), lambda i: (0, 0)),
                  pl.BlockSpec((128, 8), lambda i: (0, 0)),
                  pl.BlockSpec((8, 128), lambda i: (0, 0))],
        out_specs=[o16, o16],
        out_shape=[jax.ShapeDtypeStruct((NOCT, 128), jnp.float32),
                   jax.ShapeDtypeStruct((NOCT, 128), jnp.float32)],
    )(ea, eb, wa, wb, degrep, ba, bb, gsum, r8)


def _fin_body(p0a_ref, p1a_ref, p0b_ref, p1b_ref, w_ref, b_ref, o_ref):
    p = jnp.concatenate([p0a_ref[...] + p1a_ref[...],
                         p0b_ref[...] + p1b_ref[...]], axis=1)
    lg = jnp.dot(p, w_ref[...], preferred_element_type=jnp.float32) + b_ref[...]
    m = jnp.max(lg, axis=-1, keepdims=True)
    ex = jnp.exp(lg - m)
    o_ref[...] = ex / jnp.sum(ex, axis=-1, keepdims=True)


def _final(p0a, p1a, p0b, p1b, fcwt, fcb):
    return pl.pallas_call(
        _fin_body,
        out_shape=jax.ShapeDtypeStruct((NGRAPH, 8), jnp.float32),
    )(p0a, p1a, p0b, p1b, fcwt, fcb)


def kernel(len_y, x, index, batch, weight, lin_W, lin_b, fc_W, fc_b):
    del len_y
    row = index[0]
    col = index[1]
    f32 = jnp.float32

    wrep = jnp.broadcast_to(jnp.tile(weight, L)[:, None], (2048, FH))
    w16 = jnp.repeat(weight, L)
    zer = jnp.zeros((ACC_ROWS // NS, FH), f32)

    degrep16, dstl = _deg_call(col, wrep, zer)
    degrep = degrep16.reshape(NOCT, 128)

    wt = jnp.zeros((DP, DIN), f32).at[:HID].set(lin_W).T
    x3 = x.reshape(NOCT, 8, DIN)
    ua_o, ub_o = _prologue(x3, wt, degrep)

    def flat(a):
        return a.reshape(N, FH)

    def oct_(a):
        return a.reshape(NOCT, 128)

    e1a, e1b = _hop_call(row, dstl, w16, zer, flat(ua_o), flat(ub_o))
    wa_o, wb_o = _mid(oct_(e1a), oct_(e1b), ua_o, ub_o, degrep)
    e2a, e2b = _hop_call(row, dstl, w16, zer, flat(wa_o), flat(wb_o))

    lane = jnp.arange(128)
    ba = jnp.tile(lin_b[:FH], 8)
    bbpat = jnp.concatenate([lin_b[FH:HID], jnp.full((2,), -1e30, f32)])
    bb = jnp.tile(bbpat, 8)
    gsum = (lane[:, None] // FH == jnp.arange(8)[None, :]).astype(f32)
    r8 = gsum.T

    za_o, zb_o = _post(oct_(e2a), oct_(e2b), wa_o, wb_o, degrep,
                       ba.reshape(1, 128), bb.reshape(1, 128), gsum, r8)

    pools = _pool_call(batch, flat(za_o), flat(zb_o))
    p0a = pools[:NGRAPH]
    p0b = pools[NGRAPH:2 * NGRAPH]
    p1a = pools[2 * NGRAPH:3 * NGRAPH]
    p1b = pools[3 * NGRAPH:]

    fcw = jnp.zeros((8, DP), f32).at[:NCLS, :HID].set(fc_W)
    fcb = jnp.concatenate([fc_b, jnp.full((5,), -1e30, f32)])
    out8 = _final(p0a, p1a, p0b, p1b, fcw.T, fcb.reshape(1, 8))
    return out8[:, :NCLS]


# speedup vs baseline: 1.3706x; 1.0198x over previous
import functools

import jax
import jax.numpy as jnp
from jax import lax
from jax.experimental import pallas as pl
from jax.experimental.pallas import tpu as pltpu
from jax.experimental.pallas import tpu_sc as plsc

N = 126976
E = 262144
DIN = 128
HID = 30
DP = 32
FH = 16
NGRAPH = 2048
NCLS = 3
WP = 128

NC, NS, L = 2, 16, 16
HALF = N // 2
DUMP = 1024
ACC_ROWS = HALF + DUMP
EPT_M = E // NS
CH = 256
NCHUNK = EPT_M // CH
NOCT = N // 8

_MESH = plsc.VectorSubcoreMesh(core_axis_name="c", subcore_axis_name="s")
_SC_PARAMS = pltpu.CompilerParams(use_tc_tiling_on_sc=False)


def _zero_rows16(ref, nrows):
    zv = jnp.zeros((L,), jnp.float32)

    def body(i, _):
        ref[i, pl.ds(0, L)] = zv
        return 0

    lax.fori_loop(0, nrows, body, 0)


def _zero_acc(acc, s, zeros_hbm):
    rows_per_tile = ACC_ROWS // NS
    pltpu.sync_copy(zeros_hbm, acc.at[pl.ds(s * rows_per_tile, rows_per_tile)])


def _write_half(acc, eout, c, s, bounce):
    del bounce
    rpt = HALF // NS
    o = s * rpt
    pltpu.sync_copy(acc.at[pl.ds(o, rpt)], eout.at[pl.ds(c * HALF + o, rpt)])


DSTROWS = 2048 // CH


def _deg_body(col_hbm, wrep_hbm, zer_hbm, degrep, dstl, wrep, colbuf, dstbuf,
              dstbuf2, zbuf, acc, sem):
    c = lax.axis_index("c")
    s = lax.axis_index("s")
    lo = c * HALF

    pltpu.sync_copy(wrep_hbm, wrep)
    _zero_acc(acc, s, zer_hbm)
    plsc.subcore_barrier()

    def chunk(ch, _):
        ebase = s * EPT_M + ch * 2048
        pltpu.sync_copy(col_hbm.at[pl.ds(ebase, 2048)], colbuf)

        @plsc.parallel_loop(0, 2048 // L, unroll=2)
        def _(g):
            col_v = colbuf[pl.ds(g * L, L)]
            inhalf = (col_v >= lo) & (col_v < lo + HALF)
            dst = jnp.where(inhalf, col_v - lo, HALF + (col_v & (DUMP - 1)))
            dstbuf[pl.ds(g * L, L)] = dst
            dstbuf2[g // (CH // L), pl.ds((g % (CH // L)) * L, L)] = dst

        pltpu.sync_copy(wrep, acc.at[dstbuf], add=True)
        rowbase = (c * NS + s) * NCHUNK + ch * DSTROWS
        pltpu.sync_copy(dstbuf2, dstl.at[pl.ds(rowbase, DSTROWS)])
        return 0

    lax.fori_loop(0, EPT_M // 2048, chunk, 0)
    plsc.subcore_barrier()
    _write_half(acc, degrep, c, s, zbuf)


_deg_call = functools.partial(
    pl.kernel,
    out_type=[jax.ShapeDtypeStruct((N, FH), jnp.float32),
              jax.ShapeDtypeStruct((NC * NS * NCHUNK, CH), jnp.int32)],
    mesh=_MESH,
    compiler_params=_SC_PARAMS,
    scratch_types=[
        pltpu.VMEM((2048, FH), jnp.float32),
        pltpu.VMEM((2048,), jnp.int32),
        pltpu.VMEM((2048,), jnp.int32),
        pltpu.VMEM((DSTROWS, CH), jnp.int32),
        pltpu.VMEM((8, FH), jnp.float32),
        pltpu.VMEM_SHARED((ACC_ROWS, FH), jnp.float32),
        pltpu.SemaphoreType.DMA,
    ],
)(_deg_body)


def _hop_body(row_hbm, dstl_hbm, w16_hbm, zer_hbm, ga_hbm, gb_hbm,
              eout_a, eout_b,
              w16, rowall, dstall, gb0, gb1, gb2, gb3, acc,
              gs0, gs1, gs2, gs3, ss0, ss1, ss2, ss3):
    c = lax.axis_index("c")
    s = lax.axis_index("s")
    gbufs = (gb0, gb1, gb2, gb3)
    gsems = (gs0, gs1, gs2, gs3)
    ssems = (ss0, ss1, ss2, ss3)

    pltpu.sync_copy(w16_hbm, w16)
    ebase = s * EPT_M
    pltpu.sync_copy(row_hbm.at[pl.ds(ebase, EPT_M)], rowall)
    pltpu.sync_copy(dstl_hbm.at[pl.ds((c * NS + s) * NCHUNK, NCHUNK)], dstall)

    for fh in range(2):
        g_hbm = ga_hbm if fh == 0 else gb_hbm
        eout = eout_a if fh == 0 else eout_b

        _zero_acc(acc, s, zer_hbm)
        plsc.subcore_barrier()

        def start_gather(chv, b):
            pltpu.async_copy(g_hbm.at[rowall.at[pl.ds(chv * CH, CH)]],
                             gbufs[b], gsems[b])

        def wait_slot(sems, b):
            pltpu.make_async_copy(g_hbm.at[pl.ds(0, CH)], gbufs[b],
                                  sems[b]).wait()

        def start_scatter(chv, b):
            pltpu.async_copy(gbufs[b], acc.at[dstall.at[chv]],
                             ssems[b], add=True)

        def scale(gbuf):
            @plsc.parallel_loop(0, CH // L, unroll=4)
            def _(g):
                off = g * L
                wb = (g & 7) * (L * L)
                for k in range(L):
                    gbuf[off + k, pl.ds(0, L)] = (
                        gbuf[off + k, pl.ds(0, L)] * w16[pl.ds(wb + k * L, L)])

        start_gather(0, 0)
        start_gather(1, 1)

        def quad(q, _):
            for j in range(4):
                chv = q * 4 + j
                b2 = (j + 2) % 4
                wait_slot(gsems, j)
                scale(gbufs[j])
                start_scatter(chv, j)
                if j < 2:
                    @pl.when(q >= 1)
                    def _():
                        wait_slot(ssems, b2)
                    start_gather(chv + 2, b2)
                else:
                    wait_slot(ssems, b2)

                    @pl.when(q < NCHUNK // 4 - 1)
                    def _():
                        start_gather(chv + 2, b2)
            return 0

        lax.fori_loop(0, NCHUNK // 4, quad, 0)
        wait_slot(ssems, 2)
        wait_slot(ssems, 3)
        plsc.subcore_barrier()

        _write_half(acc, eout, c, s, gb0)
        plsc.subcore_barrier()


_hop_call = functools.partial(
    pl.kernel,
    out_type=[jax.ShapeDtypeStruct((N, FH), jnp.float32),
              jax.ShapeDtypeStruct((N, FH), jnp.float32)],
    mesh=_MESH,
    compiler_params=_SC_PARAMS,
    scratch_types=[
        pltpu.VMEM((WP * L,), jnp.float32),
        pltpu.VMEM((EPT_M,), jnp.int32),
        pltpu.VMEM((NCHUNK, CH), jnp.int32),
        pltpu.VMEM((CH, FH), jnp.float32),
        pltpu.VMEM((CH, FH), jnp.float32),
        pltpu.VMEM((CH, FH), jnp.float32),
        pltpu.VMEM((CH, FH), jnp.float32),
        pltpu.VMEM_SHARED((ACC_ROWS, FH), jnp.float32),
        pltpu.SemaphoreType.DMA,
        pltpu.SemaphoreType.DMA,
        pltpu.SemaphoreType.DMA,
        pltpu.SemaphoreType.DMA,
        pltpu.SemaphoreType.DMA,
        pltpu.SemaphoreType.DMA,
        pltpu.SemaphoreType.DMA,
        pltpu.SemaphoreType.DMA,
    ],
)(_hop_body)


def _pool_body(batch_hbm, za_hbm, zb_hbm, pout, zrow, bbuf, acc, sem):
    c = lax.axis_index("c")
    s = lax.axis_index("s")
    rpt = NGRAPH // NS
    rows = N // (NC * NS)

    for fh in range(2):
        z_hbm = za_hbm if fh == 0 else zb_hbm

        _zero_rows16(zrow, rpt)
        pltpu.sync_copy(zrow.at[pl.ds(0, rpt)], acc.at[pl.ds(s * rpt, rpt)])
        plsc.subcore_barrier()

        def chunk(ch, _):
            base = (c * NS + s) * rows + ch * 1984
            pltpu.sync_copy(batch_hbm.at[pl.ds(base, 1984)], bbuf)
            pltpu.sync_copy(z_hbm.at[pl.ds(base, 1984)], zrow)
            pltpu.sync_copy(zrow, acc.at[bbuf], add=True)
            return 0

        lax.fori_loop(0, rows // 1984, chunk, 0)
        plsc.subcore_barrier()

        pltpu.sync_copy(acc.at[pl.ds(s * rpt, rpt)], zrow.at[pl.ds(0, rpt)])
        pltpu.sync_copy(zrow.at[pl.ds(0, rpt)],
                        pout.at[pl.ds((c * 2 + fh) * NGRAPH + s * rpt, rpt)])
        plsc.subcore_barrier()


_pool_call = functools.partial(
    pl.kernel,
    out_type=jax.ShapeDtypeStruct((NC * 2 * NGRAPH, FH), jnp.float32),
    mesh=_MESH,
    compiler_params=_SC_PARAMS,
    scratch_types=[
        pltpu.VMEM((1984, FH), jnp.float32),
        pltpu.VMEM((1984,), jnp.int32),
        pltpu.VMEM_SHARED((NGRAPH, FH), jnp.float32),
        pltpu.SemaphoreType.DMA,
    ],
)(_pool_body)


OBLK = 512
NOBLK = NOCT // OBLK


def _dis_from(degrep):
    deg = degrep + 1.0
    return jnp.where(deg > 0, lax.rsqrt(deg), 0.0)


def _pro_body(x3_ref, wt_ref, dg_ref, ua_ref, ub_ref):
    wt = wt_ref[...]
    ys = [jnp.dot(x3_ref[:, j, :], wt, preferred_element_type=jnp.float32)
          for j in range(8)]
    ya = jnp.concatenate([y[:, :FH] for y in ys], axis=1)
    yb = jnp.concatenate([y[:, FH:] for y in ys], axis=1)
    dis = _dis_from(dg_ref[...])
    ua_ref[...] = ya * dis
    ub_ref[...] = yb * dis


def _prologue(x3, wt, degrep):
    o16 = pl.BlockSpec((OBLK, 128), lambda i: (i, 0))
    return pl.pallas_call(
        _pro_body,
        grid=(NOBLK,),
        in_specs=[
            pl.BlockSpec((OBLK, 8, 128), lambda i: (i, 0, 0)),
            pl.BlockSpec((DIN, DP), lambda i: (0, 0)),
            o16,
        ],
        out_specs=[o16, o16],
        out_shape=[
            jax.ShapeDtypeStruct((NOCT, 128), jnp.float32),
            jax.ShapeDtypeStruct((NOCT, 128), jnp.float32),
        ],
    )(x3, wt, degrep)


def _mid_body(ea_ref, eb_ref, ua_ref, ub_ref, dg_ref, wa_ref, wb_ref):
    deg = dg_ref[...] + 1.0
    d2 = jnp.where(deg > 0, 1.0 / deg, 0.0)
    wa_ref[...] = d2 * (ea_ref[...] + ua_ref[...])
    wb_ref[...] = d2 * (eb_ref[...] + ub_ref[...])


def _mid(ea, eb, ua, ub, degrep):
    o16 = pl.BlockSpec((OBLK, 128), lambda i: (i, 0))
    return pl.pallas_call(
        _mid_body,
        grid=(NOBLK,),
        in_specs=[o16, o16, o16, o16, o16],
        out_specs=[o16, o16],
        out_shape=[jax.ShapeDtypeStruct((NOCT, 128), jnp.float32),
                   jax.ShapeDtypeStruct((NOCT, 128), jnp.float32)],
    )(ea, eb, ua, ub, degrep)


def _post_body(ea_ref, eb_ref, wa_ref, wb_ref, dg_ref, ba_ref, bb_ref,
               gs_ref, r8_ref, za_ref, zb_ref):
    dis = _dis_from(dg_ref[...])
    ha = dis * (ea_ref[...] + wa_ref[...]) + ba_ref[...]
    hb = dis * (eb_ref[...] + wb_ref[...]) + bb_ref[...]
    exa = jnp.exp(jnp.minimum(ha, 85.0))
    exb = jnp.exp(jnp.minimum(hb, 85.0))
    s8 = (jnp.dot(exa, gs_ref[...], preferred_element_type=jnp.float32)
          + jnp.dot(exb, gs_ref[...], preferred_element_type=jnp.float32))
    rep = jnp.dot(1.0 / s8, r8_ref[...], preferred_element_type=jnp.float32)
    za_ref[...] = exa * rep
    zb_ref[...] = exb * rep


def _post(ea, eb, wa, wb, degrep, ba, bb, gsum, r8):
    o16 = pl.BlockSpec((OBLK, 128), lambda i: (i, 0))
    return pl.pallas_call(
        _post_body,
        grid=(NOBLK,),
        in_specs=[o16, o16, o16, o16, o16,
                  pl.BlockSpec((1, 128), lambda i: (0, 0)),
                  pl.BlockSpec((1, 128), lambda i: (0, 0)),
                  pl.BlockSpec((128, 8), lambda i: (0, 0)),
                  pl.BlockSpec((8, 128), lambda i: (0, 0))],
        out_specs=[o16, o16],
        out_shape=[jax.ShapeDtypeStruct((NOCT, 128), jnp.float32),
                   jax.ShapeDtypeStruct((NOCT, 128), jnp.float32)],
    )(ea, eb, wa, wb, degrep, ba, bb, gsum, r8)


def _fin_body(p0a_ref, p1a_ref, p0b_ref, p1b_ref, w_ref, b_ref, o_ref):
    p = jnp.concatenate([p0a_ref[...] + p1a_ref[...],
                         p0b_ref[...] + p1b_ref[...]], axis=1)
    lg = jnp.dot(p, w_ref[...], preferred_element_type=jnp.float32) + b_ref[...]
    m = jnp.max(lg, axis=-1, keepdims=True)
    ex = jnp.exp(lg - m)
    o_ref[...] = ex / jnp.sum(ex, axis=-1, keepdims=True)


def _final(p0a, p1a, p0b, p1b, fcwt, fcb):
    return pl.pallas_call(
        _fin_body,
        out_shape=jax.ShapeDtypeStruct((NGRAPH, 8), jnp.float32),
    )(p0a, p1a, p0b, p1b, fcwt, fcb)


def kernel(len_y, x, index, batch, weight, lin_W, lin_b, fc_W, fc_b):
    del len_y
    row = index[0]
    col = index[1]
    f32 = jnp.float32

    wrep = jnp.broadcast_to(jnp.tile(weight, L)[:, None], (2048, FH))
    w16 = jnp.repeat(weight, L)
    zer = jnp.zeros((ACC_ROWS // NS, FH), f32)

    degrep16, dstl = _deg_call(col, wrep, zer)
    degrep = degrep16.reshape(NOCT, 128)

    wt = jnp.zeros((DP, DIN), f32).at[:HID].set(lin_W).T
    x3 = x.reshape(NOCT, 8, DIN)
    ua_o, ub_o = _prologue(x3, wt, degrep)

    def flat(a):
        return a.reshape(N, FH)

    def oct_(a):
        return a.reshape(NOCT, 128)

    e1a, e1b = _hop_call(row, dstl, w16, zer, flat(ua_o), flat(ub_o))
    wa_o, wb_o = _mid(oct_(e1a), oct_(e1b), ua_o, ub_o, degrep)
    e2a, e2b = _hop_call(row, dstl, w16, zer, flat(wa_o), flat(wb_o))

    lane = jnp.arange(128)
    ba = jnp.tile(lin_b[:FH], 8)
    bbpat = jnp.concatenate([lin_b[FH:HID], jnp.full((2,), -1e30, f32)])
    bb = jnp.tile(bbpat, 8)
    gsum = (lane[:, None] // FH == jnp.arange(8)[None, :]).astype(f32)
    r8 = gsum.T

    za_o, zb_o = _post(oct_(e2a), oct_(e2b), wa_o, wb_o, degrep,
                       ba.reshape(1, 128), bb.reshape(1, 128), gsum, r8)

    pools = _pool_call(batch, flat(za_o), flat(zb_o))
    p0a = pools[:NGRAPH]
    p0b = pools[NGRAPH:2 * NGRAPH]
    p1a = pools[2 * NGRAPH:3 * NGRAPH]
    p1b = pools[3 * NGRAPH:]

    fcw = jnp.zeros((8, DP), f32).at[:NCLS, :HID].set(fc_W)
    fcb = jnp.concatenate([fc_b, jnp.full((5,), -1e30, f32)])
    out8 = _final(p0a, p1a, p0b, p1b, fcw.T, fcb.reshape(1, 8))
    return out8[:, :NCLS]


# speedup vs baseline: 1.3787x; 1.0059x over previous
import functools

import jax
import jax.numpy as jnp
from jax import lax
from jax.experimental import pallas as pl
from jax.experimental.pallas import tpu as pltpu
from jax.experimental.pallas import tpu_sc as plsc

N = 126976
E = 262144
DIN = 128
HID = 30
DP = 32
FH = 16
NGRAPH = 2048
NCLS = 3
WP = 128

NC, NS, L = 2, 16, 16
HALF = N // 2
DUMP = 1024
ACC_ROWS = HALF + DUMP
EPT_M = E // NS
CH = 256
NCHUNK = EPT_M // CH
NOCT = N // 8

_MESH = plsc.VectorSubcoreMesh(core_axis_name="c", subcore_axis_name="s")
_SC_PARAMS = pltpu.CompilerParams(use_tc_tiling_on_sc=False)


def _zero_rows16(ref, nrows):
    zv = jnp.zeros((L,), jnp.float32)

    def body(i, _):
        ref[i, pl.ds(0, L)] = zv
        return 0

    lax.fori_loop(0, nrows, body, 0)


def _zero_acc(acc, s, zeros_hbm):
    rows_per_tile = ACC_ROWS // NS
    pltpu.sync_copy(zeros_hbm, acc.at[pl.ds(s * rows_per_tile, rows_per_tile)])


def _write_half(acc, eout, c, s, bounce):
    del bounce
    rpt = HALF // NS
    o = s * rpt
    pltpu.sync_copy(acc.at[pl.ds(o, rpt)], eout.at[pl.ds(c * HALF + o, rpt)])


DSTROWS = 2048 // CH


def _deg_body(col_hbm, wrep_hbm, zer_hbm, degrep, dstl, wrep, colbuf, dstbuf,
              dstbuf2, zbuf, acc, sem):
    c = lax.axis_index("c")
    s = lax.axis_index("s")
    lo = c * HALF

    pltpu.sync_copy(wrep_hbm, wrep)
    _zero_acc(acc, s, zer_hbm)
    plsc.subcore_barrier()

    def chunk(ch, _):
        ebase = s * EPT_M + ch * 2048
        pltpu.sync_copy(col_hbm.at[pl.ds(ebase, 2048)], colbuf)

        @plsc.parallel_loop(0, 2048 // L, unroll=2)
        def _(g):
            col_v = colbuf[pl.ds(g * L, L)]
            inhalf = (col_v >= lo) & (col_v < lo + HALF)
            dst = jnp.where(inhalf, col_v - lo, HALF + (col_v & (DUMP - 1)))
            dstbuf[pl.ds(g * L, L)] = dst
            dstbuf2[g // (CH // L), pl.ds((g % (CH // L)) * L, L)] = dst

        pltpu.sync_copy(wrep, acc.at[dstbuf], add=True)
        rowbase = (c * NS + s) * NCHUNK + ch * DSTROWS
        pltpu.sync_copy(dstbuf2, dstl.at[pl.ds(rowbase, DSTROWS)])
        return 0

    lax.fori_loop(0, EPT_M // 2048, chunk, 0)
    plsc.subcore_barrier()
    _write_half(acc, degrep, c, s, zbuf)


_deg_call = functools.partial(
    pl.kernel,
    out_type=[jax.ShapeDtypeStruct((N, FH), jnp.float32),
              jax.ShapeDtypeStruct((NC * NS * NCHUNK, CH), jnp.int32)],
    mesh=_MESH,
    compiler_params=_SC_PARAMS,
    scratch_types=[
        pltpu.VMEM((2048, FH), jnp.float32),
        pltpu.VMEM((2048,), jnp.int32),
        pltpu.VMEM((2048,), jnp.int32),
        pltpu.VMEM((DSTROWS, CH), jnp.int32),
        pltpu.VMEM((8, FH), jnp.float32),
        pltpu.VMEM_SHARED((ACC_ROWS, FH), jnp.float32),
        pltpu.SemaphoreType.DMA,
    ],
)(_deg_body)


def _hop_body(row_hbm, dstl_hbm, w16_hbm, zer_hbm, ga_hbm, gb_hbm,
              eout_a, eout_b,
              w16, rowall, dstall, gb0, gb1, gb2, gb3, acc,
              gs0, gs1, gs2, gs3, ss0, ss1, ss2, ss3):
    c = lax.axis_index("c")
    s = lax.axis_index("s")
    gbufs = (gb0, gb1, gb2, gb3)
    gsems = (gs0, gs1, gs2, gs3)
    ssems = (ss0, ss1, ss2, ss3)

    pltpu.sync_copy(w16_hbm, w16)
    ebase = s * EPT_M
    pltpu.sync_copy(row_hbm.at[pl.ds(ebase, EPT_M)], rowall)
    pltpu.sync_copy(dstl_hbm.at[pl.ds((c * NS + s) * NCHUNK, NCHUNK)], dstall)

    for fh in range(2):
        g_hbm = ga_hbm if fh == 0 else gb_hbm
        eout = eout_a if fh == 0 else eout_b

        _zero_acc(acc, s, zer_hbm)
        plsc.subcore_barrier()

        def start_gather(chv, b):
            pltpu.async_copy(g_hbm.at[rowall.at[pl.ds(chv * CH, CH)]],
                             gbufs[b], gsems[b])

        def wait_slot(sems, b):
            pltpu.make_async_copy(g_hbm.at[pl.ds(0, CH)], gbufs[b],
                                  sems[b]).wait()

        def start_scatter(chv, b):
            pltpu.async_copy(gbufs[b], acc.at[dstall.at[chv]],
                             ssems[b], add=True)

        def scale(gbuf):
            @plsc.parallel_loop(0, CH // L, unroll=4)
            def _(g):
                off = g * L
                wb = (g & 7) * (L * L)
                for k in range(L):
                    gbuf[off + k, pl.ds(0, L)] = (
                        gbuf[off + k, pl.ds(0, L)] * w16[pl.ds(wb + k * L, L)])

        start_gather(0, 0)
        start_gather(1, 1)

        def quad(q, _):
            for j in range(4):
                chv = q * 4 + j
                b2 = (j + 2) % 4
                wait_slot(gsems, j)
                scale(gbufs[j])
                start_scatter(chv, j)
                if j < 2:
                    @pl.when(q >= 1)
                    def _():
                        wait_slot(ssems, b2)
                    start_gather(chv + 2, b2)
                else:
                    wait_slot(ssems, b2)

                    @pl.when(q < NCHUNK // 4 - 1)
                    def _():
                        start_gather(chv + 2, b2)
            return 0

        lax.fori_loop(0, NCHUNK // 4, quad, 0)
        wait_slot(ssems, 2)
        wait_slot(ssems, 3)
        plsc.subcore_barrier()

        _write_half(acc, eout, c, s, gb0)
        plsc.subcore_barrier()


_hop_call = functools.partial(
    pl.kernel,
    out_type=[jax.ShapeDtypeStruct((N, FH), jnp.float32),
              jax.ShapeDtypeStruct((N, FH), jnp.float32)],
    mesh=_MESH,
    compiler_params=_SC_PARAMS,
    scratch_types=[
        pltpu.VMEM((WP * L,), jnp.float32),
        pltpu.VMEM((EPT_M,), jnp.int32),
        pltpu.VMEM((NCHUNK, CH), jnp.int32),
        pltpu.VMEM((CH, FH), jnp.float32),
        pltpu.VMEM((CH, FH), jnp.float32),
        pltpu.VMEM((CH, FH), jnp.float32),
        pltpu.VMEM((CH, FH), jnp.float32),
        pltpu.VMEM_SHARED((ACC_ROWS, FH), jnp.float32),
        pltpu.SemaphoreType.DMA,
        pltpu.SemaphoreType.DMA,
        pltpu.SemaphoreType.DMA,
        pltpu.SemaphoreType.DMA,
        pltpu.SemaphoreType.DMA,
        pltpu.SemaphoreType.DMA,
        pltpu.SemaphoreType.DMA,
        pltpu.SemaphoreType.DMA,
    ],
)(_hop_body)


def _pool_body(batch_hbm, za_hbm, zb_hbm, pout, zrow, bbuf, acc, sem):
    c = lax.axis_index("c")
    s = lax.axis_index("s")
    rpt = NGRAPH // NS
    rows = N // (NC * NS)

    for fh in range(2):
        z_hbm = za_hbm if fh == 0 else zb_hbm

        _zero_rows16(zrow, rpt)
        pltpu.sync_copy(zrow.at[pl.ds(0, rpt)], acc.at[pl.ds(s * rpt, rpt)])
        plsc.subcore_barrier()

        def chunk(ch, _):
            base = (c * NS + s) * rows + ch * 1984
            pltpu.sync_copy(batch_hbm.at[pl.ds(base, 1984)], bbuf)
            pltpu.sync_copy(z_hbm.at[pl.ds(base, 1984)], zrow)
            pltpu.sync_copy(zrow, acc.at[bbuf], add=True)
            return 0

        lax.fori_loop(0, rows // 1984, chunk, 0)
        plsc.subcore_barrier()

        pltpu.sync_copy(acc.at[pl.ds(s * rpt, rpt)], zrow.at[pl.ds(0, rpt)])
        pltpu.sync_copy(zrow.at[pl.ds(0, rpt)],
                        pout.at[pl.ds((c * 2 + fh) * NGRAPH + s * rpt, rpt)])
        plsc.subcore_barrier()


_pool_call = functools.partial(
    pl.kernel,
    out_type=jax.ShapeDtypeStruct((NC * 2 * NGRAPH, FH), jnp.float32),
    mesh=_MESH,
    compiler_params=_SC_PARAMS,
    scratch_types=[
        pltpu.VMEM((1984, FH), jnp.float32),
        pltpu.VMEM((1984,), jnp.int32),
        pltpu.VMEM_SHARED((NGRAPH, FH), jnp.float32),
        pltpu.SemaphoreType.DMA,
    ],
)(_pool_body)


OBLK = 512
NOBLK = NOCT // OBLK


def _dis_from(degrep):
    deg = degrep + 1.0
    return jnp.where(deg > 0, lax.rsqrt(deg), 0.0)


def _mm_body(x3_ref, wt_ref, ya_ref, yb_ref):
    wt = wt_ref[...]
    ys = [jnp.dot(x3_ref[:, j, :], wt, preferred_element_type=jnp.float32)
          for j in range(8)]
    ya_ref[...] = jnp.concatenate([y[:, :FH] for y in ys], axis=1)
    yb_ref[...] = jnp.concatenate([y[:, FH:] for y in ys], axis=1)


def _mm(x3, wt):
    o16 = pl.BlockSpec((OBLK, 128), lambda i: (i, 0))
    return pl.pallas_call(
        _mm_body,
        grid=(NOBLK,),
        in_specs=[
            pl.BlockSpec((OBLK, 8, 128), lambda i: (i, 0, 0)),
            pl.BlockSpec((DIN, DP), lambda i: (0, 0)),
        ],
        out_specs=[o16, o16],
        out_shape=[
            jax.ShapeDtypeStruct((NOCT, 128), jnp.float32),
            jax.ShapeDtypeStruct((NOCT, 128), jnp.float32),
        ],
    )(x3, wt)


def _scl_body(ya_ref, yb_ref, dg_ref, ua_ref, ub_ref):
    dis = _dis_from(dg_ref[...])
    ua_ref[...] = ya_ref[...] * dis
    ub_ref[...] = yb_ref[...] * dis


def _scl(ya, yb, degrep):
    o16 = pl.BlockSpec((OBLK, 128), lambda i: (i, 0))
    return pl.pallas_call(
        _scl_body,
        grid=(NOBLK,),
        in_specs=[o16, o16, o16],
        out_specs=[o16, o16],
        out_shape=[
            jax.ShapeDtypeStruct((NOCT, 128), jnp.float32),
            jax.ShapeDtypeStruct((NOCT, 128), jnp.float32),
        ],
    )(ya, yb, degrep)


def _mid_body(ea_ref, eb_ref, ua_ref, ub_ref, dg_ref, wa_ref, wb_ref):
    deg = dg_ref[...] + 1.0
    d2 = jnp.where(deg > 0, 1.0 / deg, 0.0)
    wa_ref[...] = d2 * (ea_ref[...] + ua_ref[...])
    wb_ref[...] = d2 * (eb_ref[...] + ub_ref[...])


def _mid(ea, eb, ua, ub, degrep):
    o16 = pl.BlockSpec((OBLK, 128), lambda i: (i, 0))
    return pl.pallas_call(
        _mid_body,
        grid=(NOBLK,),
        in_specs=[o16, o16, o16, o16, o16],
        out_specs=[o16, o16],
        out_shape=[jax.ShapeDtypeStruct((NOCT, 128), jnp.float32),
                   jax.ShapeDtypeStruct((NOCT, 128), jnp.float32)],
    )(ea, eb, ua, ub, degrep)


def _post_body(ea_ref, eb_ref, wa_ref, wb_ref, dg_ref, ba_ref, bb_ref,
               gs_ref, r8_ref, za_ref, zb_ref):
    dis = _dis_from(dg_ref[...])
    ha = dis * (ea_ref[...] + wa_ref[...]) + ba_ref[...]
    hb = dis * (eb_ref[...] + wb_ref[...]) + bb_ref[...]
    exa = jnp.exp(jnp.minimum(ha, 85.0))
    exb = jnp.exp(jnp.minimum(hb, 85.0))
    s8 = (jnp.dot(exa, gs_ref[...], preferred_element_type=jnp.float32)
          + jnp.dot(exb, gs_ref[...], preferred_element_type=jnp.float32))
    rep = jnp.dot(1.0 / s8, r8_ref[...], preferred_element_type=jnp.float32)
    za_ref[...] = exa * rep
    zb_ref[...] = exb * rep


def _post(ea, eb, wa, wb, degrep, ba, bb, gsum, r8):
    o16 = pl.BlockSpec((OBLK, 128), lambda i: (i, 0))
    return pl.pallas_call(
        _post_body,
        grid=(NOBLK,),
        in_specs=[o16, o16, o16, o16, o16,
                  pl.BlockSpec((1, 128), lambda i: (0, 0)),
                  pl.BlockSpec((1, 128), lambda i: (0, 0)),
                  pl.BlockSpec((128, 8), lambda i: (0, 0)),
                  pl.BlockSpec((8, 128), lambda i: (0, 0))],
        out_specs=[o16, o16],
        out_shape=[jax.ShapeDtypeStruct((NOCT, 128), jnp.float32),
                   jax.ShapeDtypeStruct((NOCT, 128), jnp.float32)],
    )(ea, eb, wa, wb, degrep, ba, bb, gsum, r8)


def _fin_body(p0a_ref, p1a_ref, p0b_ref, p1b_ref, w_ref, b_ref, o_ref):
    p = jnp.concatenate([p0a_ref[...] + p1a_ref[...],
                         p0b_ref[...] + p1b_ref[...]], axis=1)
    lg = jnp.dot(p, w_ref[...], preferred_element_type=jnp.float32) + b_ref[...]
    m = jnp.max(lg, axis=-1, keepdims=True)
    ex = jnp.exp(lg - m)
    o_ref[...] = ex / jnp.sum(ex, axis=-1, keepdims=True)


def _final(p0a, p1a, p0b, p1b, fcwt, fcb):
    return pl.pallas_call(
        _fin_body,
        out_shape=jax.ShapeDtypeStruct((NGRAPH, 8), jnp.float32),
    )(p0a, p1a, p0b, p1b, fcwt, fcb)


def kernel(len_y, x, index, batch, weight, lin_W, lin_b, fc_W, fc_b):
    del len_y
    row = index[0]
    col = index[1]
    f32 = jnp.float32

    wrep = jnp.broadcast_to(jnp.tile(weight, L)[:, None], (2048, FH))
    w16 = jnp.repeat(weight, L)
    zer = jnp.zeros((ACC_ROWS // NS, FH), f32)

    wt = jnp.zeros((DP, DIN), f32).at[:HID].set(lin_W).T
    x3 = x.reshape(NOCT, 8, DIN)
    ya_o, yb_o = _mm(x3, wt)

    degrep16, dstl = _deg_call(col, wrep, zer)
    degrep = degrep16.reshape(NOCT, 128)
    ua_o, ub_o = _scl(ya_o, yb_o, degrep)

    def flat(a):
        return a.reshape(N, FH)

    def oct_(a):
        return a.reshape(NOCT, 128)

    e1a, e1b = _hop_call(row, dstl, w16, zer, flat(ua_o), flat(ub_o))
    wa_o, wb_o = _mid(oct_(e1a), oct_(e1b), ua_o, ub_o, degrep)
    e2a, e2b = _hop_call(row, dstl, w16, zer, flat(wa_o), flat(wb_o))

    lane = jnp.arange(128)
    ba = jnp.tile(lin_b[:FH], 8)
    bbpat = jnp.concatenate([lin_b[FH:HID], jnp.full((2,), -1e30, f32)])
    bb = jnp.tile(bbpat, 8)
    gsum = (lane[:, None] // FH == jnp.arange(8)[None, :]).astype(f32)
    r8 = gsum.T

    za_o, zb_o = _post(oct_(e2a), oct_(e2b), wa_o, wb_o, degrep,
                       ba.reshape(1, 128), bb.reshape(1, 128), gsum, r8)

    pools = _pool_call(batch, flat(za_o), flat(zb_o))
    p0a = pools[:NGRAPH]
    p0b = pools[NGRAPH:2 * NGRAPH]
    p1a = pools[2 * NGRAPH:3 * NGRAPH]
    p1b = pools[3 * NGRAPH:]

    fcw = jnp.zeros((8, DP), f32).at[:NCLS, :HID].set(fc_W)
    fcb = jnp.concatenate([fc_b, jnp.full((5,), -1e30, f32)])
    out8 = _final(p0a, p1a, p0b, p1b, fcw.T, fcb.reshape(1, 8))
    return out8[:, :NCLS]
